# Initial kernel scaffold; baseline (speedup 1.0000x reference)
import jax
import jax.numpy as jnp
from jax.experimental import pallas as pl


def kernel(z, edge_index, s, Wi_reset, bi_reset, Ws_reset, bs_reset, Wi_update, bi_update, Ws_update, bs_update, Wi_cand, bi_cand, Ws_cand, bs_cand):
    raise NotImplementedError("write your pallas kernel here")


# speedup vs baseline: 17.6140x; 17.6140x over previous
import functools

import jax
import jax.numpy as jnp
from jax import lax
from jax.experimental import pallas as pl
from jax.experimental.pallas import tpu as pltpu
from jax.experimental.pallas import tpu_sc as plsc

N = 10000
E = 320000
DZ = 256
DS = 128
DF = 96
CHUNK = 80
NCH_DEG = E // (32 * CHUNK)
NCH_AGG = E // (16 * CHUNK)
NP = 10240
DEG_SL = 640
ROW_SL = 632
ROW_SL_LAST = N - 15 * ROW_SL


def _deg_body(dst_hbm, out_hbm, dst_v, ones_v, zeros_v, deg_sp):
    c = lax.axis_index("c")
    s = lax.axis_index("s")
    wid = c * 16 + s

    def fill(i, _):
        ones_v[pl.ds(i * 16, 16)] = jnp.full((16,), 1.0, jnp.float32)
        zeros_v[pl.ds(i * 16, 16)] = jnp.zeros((16,), jnp.float32)
        return 0
    lax.fori_loop(0, 5, fill, 0)

    def fillz(i, _):
        zeros_v[pl.ds(i * 16, 16)] = jnp.zeros((16,), jnp.float32)
        return 0
    lax.fori_loop(5, 40, fillz, 0)

    pltpu.sync_copy(zeros_v, deg_sp.at[pl.ds(s * DEG_SL, DEG_SL)])
    pltpu.sync_copy(dst_hbm.at[wid], dst_v)
    plsc.subcore_barrier()

    def body(j, _):
        pltpu.sync_copy(ones_v, deg_sp.at[dst_v.at[j]], add=True)
        return 0
    lax.fori_loop(0, NCH_DEG, body, 0)
    plsc.subcore_barrier()

    @pl.when(c == 0)
    def _():
        pltpu.sync_copy(deg_sp.at[pl.ds(s * DEG_SL, DEG_SL)],
                        out_hbm.at[0, 0, pl.ds(s * DEG_SL, DEG_SL)])

    @pl.when(c == 1)
    def _():
        pltpu.sync_copy(deg_sp.at[pl.ds(s * DEG_SL, DEG_SL)],
                        out_hbm.at[1, 0, pl.ds(s * DEG_SL, DEG_SL)])


def _agg_body(xs0, xs1, xs2, xs3, src_hbm, dst_hbm,
              out0, out1, out2, out3,
              src_v, dst_v, rows_v, zrows_v, agg_sp):
    c = lax.axis_index("c")
    s = lax.axis_index("s")

    def zrow(i, _):
        def zcol(k, _2):
            zrows_v[i, pl.ds(k * 16, 16)] = jnp.zeros((16,), jnp.float32)
            return 0
        lax.fori_loop(0, DF // 16, zcol, 0)
        return 0
    lax.fori_loop(0, CHUNK, zrow, 0)

    pltpu.sync_copy(src_hbm.at[s], src_v)
    pltpu.sync_copy(dst_hbm.at[s], dst_v)

    row0 = s * ROW_SL
    tables = (xs0, xs1, xs2, xs3)
    outs = (out0, out1, out2, out3)

    def zero_mine():
        @pl.when(s < 15)
        def _():
            for t in range(ROW_SL // CHUNK):
                pltpu.sync_copy(zrows_v,
                                agg_sp.at[pl.ds(row0 + t * CHUNK, CHUNK)])
            rem = ROW_SL % CHUNK
            pltpu.sync_copy(zrows_v.at[pl.ds(0, rem)],
                            agg_sp.at[pl.ds(row0 + ROW_SL - rem, rem)])

        @pl.when(s == 15)
        def _():
            base = 15 * ROW_SL
            for t in range(ROW_SL_LAST // CHUNK):
                pltpu.sync_copy(zrows_v,
                                agg_sp.at[pl.ds(base + t * CHUNK, CHUNK)])
            rem = ROW_SL_LAST % CHUNK
            pltpu.sync_copy(zrows_v.at[pl.ds(0, rem)],
                            agg_sp.at[pl.ds(base + ROW_SL_LAST - rem, rem)])

    def scatter_all(table):
        def body(j, _):
            pltpu.sync_copy(table.at[src_v.at[j]], rows_v)
            pltpu.sync_copy(rows_v, agg_sp.at[dst_v.at[j]], add=True)
            return 0
        lax.fori_loop(0, NCH_AGG, body, 0)

    def dump_mine(out):
        @pl.when(s < 15)
        def _():
            pltpu.sync_copy(agg_sp.at[pl.ds(row0, ROW_SL)],
                            out.at[pl.ds(row0, ROW_SL)])

        @pl.when(s == 15)
        def _():
            pltpu.sync_copy(agg_sp.at[pl.ds(15 * ROW_SL, ROW_SL_LAST)],
                            out.at[pl.ds(15 * ROW_SL, ROW_SL_LAST)])

    for p in range(2):
        zero_mine()
        plsc.subcore_barrier()

        @pl.when(c == 0)
        def _(p=p):
            scatter_all(tables[2 * p])

        @pl.when(c == 1)
        def _(p=p):
            scatter_all(tables[2 * p + 1])

        plsc.subcore_barrier()

        @pl.when(c == 0)
        def _(p=p):
            dump_mine(outs[2 * p])

        @pl.when(c == 1)
        def _(p=p):
            dump_mine(outs[2 * p + 1])

        if p == 0:
            plsc.subcore_barrier()


@functools.cache
def _sc_kernels():
    mesh = plsc.VectorSubcoreMesh(core_axis_name="c", subcore_axis_name="s")
    deg_k = functools.partial(
        pl.kernel,
        out_type=jax.ShapeDtypeStruct((2, 1, NP), jnp.float32),
        mesh=mesh,
        scratch_types=[
            pltpu.VMEM((NCH_DEG, CHUNK), jnp.int32),
            pltpu.VMEM((CHUNK,), jnp.float32),
            pltpu.VMEM((DEG_SL,), jnp.float32),
            pltpu.VMEM_SHARED((NP,), jnp.float32),
        ],
    )(_deg_body)
    agg_k = functools.partial(
        pl.kernel,
        out_type=[jax.ShapeDtypeStruct((N, DF), jnp.float32)] * 4,
        mesh=mesh,
        compiler_params=pltpu.CompilerParams(use_tc_tiling_on_sc=False),
        scratch_types=[
            pltpu.VMEM((NCH_AGG, CHUNK), jnp.int32),
            pltpu.VMEM((NCH_AGG, CHUNK), jnp.int32),
            pltpu.VMEM((CHUNK, DF), jnp.float32),
            pltpu.VMEM((CHUNK, DF), jnp.float32),
            pltpu.VMEM_SHARED((N, DF), jnp.float32),
        ],
    )(_agg_body)
    return deg_k, agg_k


def _prescale_body(d0, d1, z_ref, s_ref, xs0, xs1, xs2, xs3,
                   dinv_ref, selfc_ref):
    deg = d0[...] + d1[...] + 2.0
    di = lax.rsqrt(deg)
    dinv_ref[...] = di
    selfc_ref[...] = 2.0 / deg
    zs = z_ref[...] * di
    ss = s_ref[...] * di
    xs0[...] = zs[:, :96]
    xs1[...] = zs[:, 96:192]
    xs2[:, :64] = zs[:, 192:]
    xs2[:, 64:] = ss[:, :32]
    xs3[...] = ss[:, 32:]


def _prescale(d0, d1, z, s):
    R = 2000
    grid = N // R
    return pl.pallas_call(
        _prescale_body,
        grid=(grid,),
        in_specs=[
            pl.BlockSpec((R, 1), lambda i: (i, 0)),
            pl.BlockSpec((R, 1), lambda i: (i, 0)),
            pl.BlockSpec((R, DZ), lambda i: (i, 0)),
            pl.BlockSpec((R, DS), lambda i: (i, 0)),
        ],
        out_specs=[pl.BlockSpec((R, DF), lambda i: (i, 0))] * 4
        + [pl.BlockSpec((R, 1), lambda i: (i, 0))] * 2,
        out_shape=[jax.ShapeDtypeStruct((N, DF), jnp.float32)] * 4
        + [jax.ShapeDtypeStruct((N, 1), jnp.float32)] * 2,
    )(d0, d1, z, s)


def _final_body(a0, a1, a2, a3, z_ref, s_ref, dinv_ref, selfc_ref,
                wi_ref, ws_ref, bi_ref, bs_ref, out_ref):
    di = dinv_ref[...]
    sc = selfc_ref[...]
    g0 = di * a0[...]
    g1 = di * a1[...]
    g2 = di * a2[...]
    g3 = di * a3[...]
    wi = wi_ref[...]
    ws = ws_ref[...]
    f32 = jnp.float32
    gz = (jnp.dot(sc * z_ref[...], wi, preferred_element_type=f32)
          + jnp.dot(g0, wi[:96, :], preferred_element_type=f32)
          + jnp.dot(g1, wi[96:192, :], preferred_element_type=f32)
          + jnp.dot(g2[:, :64], wi[192:, :], preferred_element_type=f32)
          + bi_ref[...])
    gs = (jnp.dot(sc * s_ref[...], ws, preferred_element_type=f32)
          + jnp.dot(g2[:, 64:], ws[:32, :], preferred_element_type=f32)
          + jnp.dot(g3, ws[32:, :], preferred_element_type=f32)
          + bs_ref[...])
    reset = jax.nn.sigmoid(gz[:, :128] + gs[:, :128])
    update = jax.nn.sigmoid(gz[:, 128:256] + gs[:, 128:256])
    cand = jnp.tanh(gz[:, 256:] + reset * gs[:, 256:])
    out_ref[...] = (1.0 - update) * cand + update * s_ref[...]


def _final(a0, a1, a2, a3, z, s, dinv, selfc, wi, ws, bi, bs):
    R = 2000
    grid = N // R
    return pl.pallas_call(
        _final_body,
        grid=(grid,),
        in_specs=[
            pl.BlockSpec((R, DF), lambda i: (i, 0)),
            pl.BlockSpec((R, DF), lambda i: (i, 0)),
            pl.BlockSpec((R, DF), lambda i: (i, 0)),
            pl.BlockSpec((R, DF), lambda i: (i, 0)),
            pl.BlockSpec((R, DZ), lambda i: (i, 0)),
            pl.BlockSpec((R, DS), lambda i: (i, 0)),
            pl.BlockSpec((R, 1), lambda i: (i, 0)),
            pl.BlockSpec((R, 1), lambda i: (i, 0)),
            pl.BlockSpec((DZ, 3 * DS), lambda i: (0, 0)),
            pl.BlockSpec((DS, 3 * DS), lambda i: (0, 0)),
            pl.BlockSpec((1, 3 * DS), lambda i: (0, 0)),
            pl.BlockSpec((1, 3 * DS), lambda i: (0, 0)),
        ],
        out_specs=pl.BlockSpec((R, DS), lambda i: (i, 0)),
        out_shape=jax.ShapeDtypeStruct((N, DS), jnp.float32),
    )(a0, a1, a2, a3, z, s, dinv, selfc, wi, ws, bi, bs)


def kernel(z, edge_index, s,
           Wi_reset, bi_reset, Ws_reset, bs_reset,
           Wi_update, bi_update, Ws_update, bs_update,
           Wi_cand, bi_cand, Ws_cand, bs_cand):
    src = edge_index[0]
    dst = edge_index[1]
    dst_deg = dst.reshape(32, NCH_DEG, CHUNK)
    src_agg = src.reshape(16, NCH_AGG, CHUNK)
    dst_agg = dst.reshape(16, NCH_AGG, CHUNK)

    deg_kernel, agg_kernel = _sc_kernels()
    deg_parts = deg_kernel(dst_deg)
    d0 = deg_parts[0, 0, :N].reshape(N, 1)
    d1 = deg_parts[1, 0, :N].reshape(N, 1)

    xs0, xs1, xs2, xs3, dinv, selfc = _prescale(d0, d1, z, s)
    a0, a1, a2, a3 = agg_kernel(xs0, xs1, xs2, xs3, src_agg, dst_agg)

    wi = jnp.concatenate([Wi_reset, Wi_update, Wi_cand], axis=1)
    ws = jnp.concatenate([Ws_reset, Ws_update, Ws_cand], axis=1)
    bi = jnp.concatenate([bi_reset, bi_update, bi_cand]).reshape(1, 3 * DS)
    bs = jnp.concatenate([bs_reset, bs_update, bs_cand]).reshape(1, 3 * DS)

    return _final(a0, a1, a2, a3, z, s, dinv, selfc, wi, ws, bi, bs)


# speedup vs baseline: 22.4806x; 1.2763x over previous
import functools

import jax
import jax.numpy as jnp
from jax import lax
from jax.experimental import pallas as pl
from jax.experimental.pallas import tpu as pltpu
from jax.experimental.pallas import tpu_sc as plsc

N = 10000
E = 320000
DZ = 256
DS = 128
DF = 96
CHUNK = 80
NCH_DEG = E // (32 * CHUNK)
NCH_AGG = E // (16 * CHUNK)
NP = 10240
DEG_SL = 640
ROW_SL = 632
ROW_SL_LAST = N - 15 * ROW_SL


def _deg_body(dst_hbm, out_hbm, dst_v, ones_v, zeros_v, deg_sp):
    c = lax.axis_index("c")
    s = lax.axis_index("s")
    wid = c * 16 + s

    def fill(i, _):
        ones_v[pl.ds(i * 16, 16)] = jnp.full((16,), 1.0, jnp.float32)
        zeros_v[pl.ds(i * 16, 16)] = jnp.zeros((16,), jnp.float32)
        return 0
    lax.fori_loop(0, 5, fill, 0)

    def fillz(i, _):
        zeros_v[pl.ds(i * 16, 16)] = jnp.zeros((16,), jnp.float32)
        return 0
    lax.fori_loop(5, 40, fillz, 0)

    pltpu.sync_copy(zeros_v, deg_sp.at[pl.ds(s * DEG_SL, DEG_SL)])
    pltpu.sync_copy(dst_hbm.at[wid], dst_v)
    plsc.subcore_barrier()

    def body(j, _):
        pltpu.sync_copy(ones_v, deg_sp.at[dst_v.at[j]], add=True)
        return 0
    lax.fori_loop(0, NCH_DEG, body, 0)
    plsc.subcore_barrier()

    @pl.when(c == 0)
    def _():
        pltpu.sync_copy(deg_sp.at[pl.ds(s * DEG_SL, DEG_SL)],
                        out_hbm.at[0, 0, pl.ds(s * DEG_SL, DEG_SL)])

    @pl.when(c == 1)
    def _():
        pltpu.sync_copy(deg_sp.at[pl.ds(s * DEG_SL, DEG_SL)],
                        out_hbm.at[1, 0, pl.ds(s * DEG_SL, DEG_SL)])


def _agg_body(xs0, xs1, xs2, xs3, src_hbm, dst_hbm,
              out0, out1, out2, out3,
              src_v, dst_v, rows_v, rows2_v, zrows_v, sem0, sem1, agg_sp):
    c = lax.axis_index("c")
    s = lax.axis_index("s")

    def zrow(i, _):
        def zcol(k, _2):
            zrows_v[i, pl.ds(k * 16, 16)] = jnp.zeros((16,), jnp.float32)
            return 0
        lax.fori_loop(0, DF // 16, zcol, 0)
        return 0
    lax.fori_loop(0, CHUNK, zrow, 0)

    pltpu.sync_copy(src_hbm.at[s], src_v)
    pltpu.sync_copy(dst_hbm.at[s], dst_v)

    row0 = s * ROW_SL
    tables = (xs0, xs1, xs2, xs3)
    outs = (out0, out1, out2, out3)

    def zero_mine():
        @pl.when(s < 15)
        def _():
            for t in range(ROW_SL // CHUNK):
                pltpu.sync_copy(zrows_v,
                                agg_sp.at[pl.ds(row0 + t * CHUNK, CHUNK)])
            rem = ROW_SL % CHUNK
            pltpu.sync_copy(zrows_v.at[pl.ds(0, rem)],
                            agg_sp.at[pl.ds(row0 + ROW_SL - rem, rem)])

        @pl.when(s == 15)
        def _():
            base = 15 * ROW_SL
            for t in range(ROW_SL_LAST // CHUNK):
                pltpu.sync_copy(zrows_v,
                                agg_sp.at[pl.ds(base + t * CHUNK, CHUNK)])
            rem = ROW_SL_LAST % CHUNK
            pltpu.sync_copy(zrows_v.at[pl.ds(0, rem)],
                            agg_sp.at[pl.ds(base + ROW_SL_LAST - rem, rem)])

    def scatter_all(table):
        def body(jj, _):
            j0 = 2 * jj
            h0 = pltpu.async_copy(table.at[src_v.at[j0]], rows_v, sem0)
            h1 = pltpu.async_copy(table.at[src_v.at[j0 + 1]], rows2_v, sem1)
            h0.wait()
            pltpu.sync_copy(rows_v, agg_sp.at[dst_v.at[j0]], add=True)
            h1.wait()
            pltpu.sync_copy(rows2_v, agg_sp.at[dst_v.at[j0 + 1]], add=True)
            return 0
        lax.fori_loop(0, NCH_AGG // 2, body, 0)

    def dump_mine(out):
        @pl.when(s < 15)
        def _():
            pltpu.sync_copy(agg_sp.at[pl.ds(row0, ROW_SL)],
                            out.at[pl.ds(row0, ROW_SL)])

        @pl.when(s == 15)
        def _():
            pltpu.sync_copy(agg_sp.at[pl.ds(15 * ROW_SL, ROW_SL_LAST)],
                            out.at[pl.ds(15 * ROW_SL, ROW_SL_LAST)])

    for p in range(2):
        zero_mine()
        plsc.subcore_barrier()

        @pl.when(c == 0)
        def _(p=p):
            scatter_all(tables[2 * p])

        @pl.when(c == 1)
        def _(p=p):
            scatter_all(tables[2 * p + 1])

        plsc.subcore_barrier()

        @pl.when(c == 0)
        def _(p=p):
            dump_mine(outs[2 * p])

        @pl.when(c == 1)
        def _(p=p):
            dump_mine(outs[2 * p + 1])

        if p == 0:
            plsc.subcore_barrier()


@functools.cache
def _sc_kernels():
    mesh = plsc.VectorSubcoreMesh(core_axis_name="c", subcore_axis_name="s")
    deg_k = functools.partial(
        pl.kernel,
        out_type=jax.ShapeDtypeStruct((2, 1, NP), jnp.float32),
        mesh=mesh,
        scratch_types=[
            pltpu.VMEM((NCH_DEG, CHUNK), jnp.int32),
            pltpu.VMEM((CHUNK,), jnp.float32),
            pltpu.VMEM((DEG_SL,), jnp.float32),
            pltpu.VMEM_SHARED((NP,), jnp.float32),
        ],
    )(_deg_body)
    agg_k = functools.partial(
        pl.kernel,
        out_type=[jax.ShapeDtypeStruct((N, DF), jnp.float32)] * 4,
        mesh=mesh,
        compiler_params=pltpu.CompilerParams(use_tc_tiling_on_sc=False),
        scratch_types=[
            pltpu.VMEM((NCH_AGG, CHUNK), jnp.int32),
            pltpu.VMEM((NCH_AGG, CHUNK), jnp.int32),
            pltpu.VMEM((CHUNK, DF), jnp.float32),
            pltpu.VMEM((CHUNK, DF), jnp.float32),
            pltpu.VMEM((CHUNK, DF), jnp.float32),
            pltpu.SemaphoreType.DMA,
            pltpu.SemaphoreType.DMA,
            pltpu.VMEM_SHARED((N, DF), jnp.float32),
        ],
    )(_agg_body)
    return deg_k, agg_k


def _prescale_body(d0, d1, z_ref, s_ref, xs0, xs1, xs2, xs3,
                   dinv_ref, selfc_ref):
    deg = d0[...] + d1[...] + 2.0
    di = lax.rsqrt(deg)
    dinv_ref[...] = di
    selfc_ref[...] = 2.0 / deg
    zs = z_ref[...] * di
    ss = s_ref[...] * di
    xs0[...] = zs[:, :96]
    xs1[...] = zs[:, 96:192]
    xs2[:, :64] = zs[:, 192:]
    xs2[:, 64:] = ss[:, :32]
    xs3[...] = ss[:, 32:]


def _prescale(d0, d1, z, s):
    R = 2000
    grid = N // R
    return pl.pallas_call(
        _prescale_body,
        grid=(grid,),
        in_specs=[
            pl.BlockSpec((R, 1), lambda i: (i, 0)),
            pl.BlockSpec((R, 1), lambda i: (i, 0)),
            pl.BlockSpec((R, DZ), lambda i: (i, 0)),
            pl.BlockSpec((R, DS), lambda i: (i, 0)),
        ],
        out_specs=[pl.BlockSpec((R, DF), lambda i: (i, 0))] * 4
        + [pl.BlockSpec((R, 1), lambda i: (i, 0))] * 2,
        out_shape=[jax.ShapeDtypeStruct((N, DF), jnp.float32)] * 4
        + [jax.ShapeDtypeStruct((N, 1), jnp.float32)] * 2,
    )(d0, d1, z, s)


def _final_body(a0, a1, a2, a3, z_ref, s_ref, dinv_ref, selfc_ref,
                wi_ref, ws_ref, bi_ref, bs_ref, out_ref):
    di = dinv_ref[...]
    sc = selfc_ref[...]
    g0 = di * a0[...]
    g1 = di * a1[...]
    g2 = di * a2[...]
    g3 = di * a3[...]
    wi = wi_ref[...]
    ws = ws_ref[...]
    f32 = jnp.float32
    gz = (jnp.dot(sc * z_ref[...], wi, preferred_element_type=f32)
          + jnp.dot(g0, wi[:96, :], preferred_element_type=f32)
          + jnp.dot(g1, wi[96:192, :], preferred_element_type=f32)
          + jnp.dot(g2[:, :64], wi[192:, :], preferred_element_type=f32)
          + bi_ref[...])
    gs = (jnp.dot(sc * s_ref[...], ws, preferred_element_type=f32)
          + jnp.dot(g2[:, 64:], ws[:32, :], preferred_element_type=f32)
          + jnp.dot(g3, ws[32:, :], preferred_element_type=f32)
          + bs_ref[...])
    reset = jax.nn.sigmoid(gz[:, :128] + gs[:, :128])
    update = jax.nn.sigmoid(gz[:, 128:256] + gs[:, 128:256])
    cand = jnp.tanh(gz[:, 256:] + reset * gs[:, 256:])
    out_ref[...] = (1.0 - update) * cand + update * s_ref[...]


def _final(a0, a1, a2, a3, z, s, dinv, selfc, wi, ws, bi, bs):
    R = 2000
    grid = N // R
    return pl.pallas_call(
        _final_body,
        grid=(grid,),
        in_specs=[
            pl.BlockSpec((R, DF), lambda i: (i, 0)),
            pl.BlockSpec((R, DF), lambda i: (i, 0)),
            pl.BlockSpec((R, DF), lambda i: (i, 0)),
            pl.BlockSpec((R, DF), lambda i: (i, 0)),
            pl.BlockSpec((R, DZ), lambda i: (i, 0)),
            pl.BlockSpec((R, DS), lambda i: (i, 0)),
            pl.BlockSpec((R, 1), lambda i: (i, 0)),
            pl.BlockSpec((R, 1), lambda i: (i, 0)),
            pl.BlockSpec((DZ, 3 * DS), lambda i: (0, 0)),
            pl.BlockSpec((DS, 3 * DS), lambda i: (0, 0)),
            pl.BlockSpec((1, 3 * DS), lambda i: (0, 0)),
            pl.BlockSpec((1, 3 * DS), lambda i: (0, 0)),
        ],
        out_specs=pl.BlockSpec((R, DS), lambda i: (i, 0)),
        out_shape=jax.ShapeDtypeStruct((N, DS), jnp.float32),
    )(a0, a1, a2, a3, z, s, dinv, selfc, wi, ws, bi, bs)


def kernel(z, edge_index, s,
           Wi_reset, bi_reset, Ws_reset, bs_reset,
           Wi_update, bi_update, Ws_update, bs_update,
           Wi_cand, bi_cand, Ws_cand, bs_cand):
    src = edge_index[0]
    dst = edge_index[1]
    dst_deg = dst.reshape(32, NCH_DEG, CHUNK)
    src_agg = src.reshape(16, NCH_AGG, CHUNK)
    dst_agg = dst.reshape(16, NCH_AGG, CHUNK)

    deg_kernel, agg_kernel = _sc_kernels()
    deg_parts = deg_kernel(dst_deg)
    d0 = deg_parts[0, 0, :N].reshape(N, 1)
    d1 = deg_parts[1, 0, :N].reshape(N, 1)

    xs0, xs1, xs2, xs3, dinv, selfc = _prescale(d0, d1, z, s)
    a0, a1, a2, a3 = agg_kernel(xs0, xs1, xs2, xs3, src_agg, dst_agg)

    wi = jnp.concatenate([Wi_reset, Wi_update, Wi_cand], axis=1)
    ws = jnp.concatenate([Ws_reset, Ws_update, Ws_cand], axis=1)
    bi = jnp.concatenate([bi_reset, bi_update, bi_cand]).reshape(1, 3 * DS)
    bs = jnp.concatenate([bs_reset, bs_update, bs_cand]).reshape(1, 3 * DS)

    return _final(a0, a1, a2, a3, z, s, dinv, selfc, wi, ws, bi, bs)


# speedup vs baseline: 22.5889x; 1.0048x over previous
import functools

import jax
import jax.numpy as jnp
from jax import lax
from jax.experimental import pallas as pl
from jax.experimental.pallas import tpu as pltpu
from jax.experimental.pallas import tpu_sc as plsc

N = 10000
E = 320000
DZ = 256
DS = 128
DF = 96
CHUNK = 80
NCH_DEG = E // (32 * CHUNK)
ACH = 50
NBUF = 4
NCH_AGG = E // (16 * ACH)
NP = 10240
DEG_SL = 640
ROW_SL = 632
ROW_SL_LAST = N - 15 * ROW_SL


def _deg_body(dst_hbm, out_hbm, dst_v, ones_v, zeros_v, deg_sp):
    c = lax.axis_index("c")
    s = lax.axis_index("s")
    wid = c * 16 + s

    def fill(i, _):
        ones_v[pl.ds(i * 16, 16)] = jnp.full((16,), 1.0, jnp.float32)
        zeros_v[pl.ds(i * 16, 16)] = jnp.zeros((16,), jnp.float32)
        return 0
    lax.fori_loop(0, 5, fill, 0)

    def fillz(i, _):
        zeros_v[pl.ds(i * 16, 16)] = jnp.zeros((16,), jnp.float32)
        return 0
    lax.fori_loop(5, 40, fillz, 0)

    pltpu.sync_copy(zeros_v, deg_sp.at[pl.ds(s * DEG_SL, DEG_SL)])
    pltpu.sync_copy(dst_hbm.at[wid], dst_v)
    plsc.subcore_barrier()

    def body(j, _):
        pltpu.sync_copy(ones_v, deg_sp.at[dst_v.at[j]], add=True)
        return 0
    lax.fori_loop(0, NCH_DEG, body, 0)
    plsc.subcore_barrier()

    @pl.when(c == 0)
    def _():
        pltpu.sync_copy(deg_sp.at[pl.ds(s * DEG_SL, DEG_SL)],
                        out_hbm.at[0, 0, pl.ds(s * DEG_SL, DEG_SL)])

    @pl.when(c == 1)
    def _():
        pltpu.sync_copy(deg_sp.at[pl.ds(s * DEG_SL, DEG_SL)],
                        out_hbm.at[1, 0, pl.ds(s * DEG_SL, DEG_SL)])


def _agg_body(xs0, xs1, xs2, xs3, src_hbm, dst_hbm,
              out0, out1, out2, out3,
              src_v, dst_v, b0, b1, b2, b3, zrows_v,
              s0, s1, s2, s3, agg_sp):
    c = lax.axis_index("c")
    s = lax.axis_index("s")
    bufs = (b0, b1, b2, b3)
    sems = (s0, s1, s2, s3)

    def zrow(i, _):
        def zcol(k, _2):
            zrows_v[i, pl.ds(k * 16, 16)] = jnp.zeros((16,), jnp.float32)
            return 0
        lax.fori_loop(0, DF // 16, zcol, 0)
        return 0
    lax.fori_loop(0, ACH, zrow, 0)

    pltpu.sync_copy(src_hbm.at[s], src_v)
    pltpu.sync_copy(dst_hbm.at[s], dst_v)

    row0 = s * ROW_SL
    tables = (xs0, xs1, xs2, xs3)
    outs = (out0, out1, out2, out3)

    def zero_mine():
        @pl.when(s < 15)
        def _():
            for t in range(ROW_SL // ACH):
                pltpu.sync_copy(zrows_v,
                                agg_sp.at[pl.ds(row0 + t * ACH, ACH)])
            rem = ROW_SL % ACH
            pltpu.sync_copy(zrows_v.at[pl.ds(0, rem)],
                            agg_sp.at[pl.ds(row0 + ROW_SL - rem, rem)])

        @pl.when(s == 15)
        def _():
            base = 15 * ROW_SL
            for t in range(ROW_SL_LAST // ACH):
                pltpu.sync_copy(zrows_v,
                                agg_sp.at[pl.ds(base + t * ACH, ACH)])
            rem = ROW_SL_LAST % ACH
            pltpu.sync_copy(zrows_v.at[pl.ds(0, rem)],
                            agg_sp.at[pl.ds(base + ROW_SL_LAST - rem, rem)])

    def scatter_all(table):
        def body(g, _):
            base = g * NBUF
            hs = [pltpu.async_copy(table.at[src_v.at[base + i]],
                                   bufs[i], sems[i])
                  for i in range(NBUF)]
            for i in range(NBUF):
                hs[i].wait()
                pltpu.sync_copy(bufs[i], agg_sp.at[dst_v.at[base + i]],
                                add=True)
            return 0
        lax.fori_loop(0, NCH_AGG // NBUF, body, 0)

    def dump_mine(out):
        @pl.when(s < 15)
        def _():
            pltpu.sync_copy(agg_sp.at[pl.ds(row0, ROW_SL)],
                            out.at[pl.ds(row0, ROW_SL)])

        @pl.when(s == 15)
        def _():
            pltpu.sync_copy(agg_sp.at[pl.ds(15 * ROW_SL, ROW_SL_LAST)],
                            out.at[pl.ds(15 * ROW_SL, ROW_SL_LAST)])

    for p in range(2):
        zero_mine()
        plsc.subcore_barrier()

        @pl.when(c == 0)
        def _(p=p):
            scatter_all(tables[2 * p])

        @pl.when(c == 1)
        def _(p=p):
            scatter_all(tables[2 * p + 1])

        plsc.subcore_barrier()

        @pl.when(c == 0)
        def _(p=p):
            dump_mine(outs[2 * p])

        @pl.when(c == 1)
        def _(p=p):
            dump_mine(outs[2 * p + 1])

        if p == 0:
            plsc.subcore_barrier()


@functools.cache
def _sc_kernels():
    mesh = plsc.VectorSubcoreMesh(core_axis_name="c", subcore_axis_name="s")
    deg_k = functools.partial(
        pl.kernel,
        out_type=jax.ShapeDtypeStruct((2, 1, NP), jnp.float32),
        mesh=mesh,
        scratch_types=[
            pltpu.VMEM((NCH_DEG, CHUNK), jnp.int32),
            pltpu.VMEM((CHUNK,), jnp.float32),
            pltpu.VMEM((DEG_SL,), jnp.float32),
            pltpu.VMEM_SHARED((NP,), jnp.float32),
        ],
    )(_deg_body)
    agg_k = functools.partial(
        pl.kernel,
        out_type=[jax.ShapeDtypeStruct((N, DF), jnp.float32)] * 4,
        mesh=mesh,
        compiler_params=pltpu.CompilerParams(use_tc_tiling_on_sc=False),
        scratch_types=[
            pltpu.VMEM((NCH_AGG, ACH), jnp.int32),
            pltpu.VMEM((NCH_AGG, ACH), jnp.int32),
        ] + [pltpu.VMEM((ACH, DF), jnp.float32)] * (NBUF + 1)
        + [pltpu.SemaphoreType.DMA] * NBUF
        + [pltpu.VMEM_SHARED((N, DF), jnp.float32)],
    )(_agg_body)
    return deg_k, agg_k


def _prescale_body(d0, d1, z_ref, s_ref, xs0, xs1, xs2, xs3,
                   dinv_ref, selfc_ref):
    deg = d0[...] + d1[...] + 2.0
    di = lax.rsqrt(deg)
    dinv_ref[...] = di
    selfc_ref[...] = 2.0 / deg
    zs = z_ref[...] * di
    ss = s_ref[...] * di
    xs0[...] = zs[:, :96]
    xs1[...] = zs[:, 96:192]
    xs2[:, :64] = zs[:, 192:]
    xs2[:, 64:] = ss[:, :32]
    xs3[...] = ss[:, 32:]


def _prescale(d0, d1, z, s):
    R = 2000
    grid = N // R
    return pl.pallas_call(
        _prescale_body,
        grid=(grid,),
        in_specs=[
            pl.BlockSpec((R, 1), lambda i: (i, 0)),
            pl.BlockSpec((R, 1), lambda i: (i, 0)),
            pl.BlockSpec((R, DZ), lambda i: (i, 0)),
            pl.BlockSpec((R, DS), lambda i: (i, 0)),
        ],
        out_specs=[pl.BlockSpec((R, DF), lambda i: (i, 0))] * 4
        + [pl.BlockSpec((R, 1), lambda i: (i, 0))] * 2,
        out_shape=[jax.ShapeDtypeStruct((N, DF), jnp.float32)] * 4
        + [jax.ShapeDtypeStruct((N, 1), jnp.float32)] * 2,
    )(d0, d1, z, s)


def _final_body(a0, a1, a2, a3, z_ref, s_ref, dinv_ref, selfc_ref,
                wi_ref, ws_ref, bi_ref, bs_ref, out_ref):
    di = dinv_ref[...]
    sc = selfc_ref[...]
    g0 = di * a0[...]
    g1 = di * a1[...]
    g2 = di * a2[...]
    g3 = di * a3[...]
    wi = wi_ref[...]
    ws = ws_ref[...]
    f32 = jnp.float32
    gz = (jnp.dot(sc * z_ref[...], wi, preferred_element_type=f32)
          + jnp.dot(g0, wi[:96, :], preferred_element_type=f32)
          + jnp.dot(g1, wi[96:192, :], preferred_element_type=f32)
          + jnp.dot(g2[:, :64], wi[192:, :], preferred_element_type=f32)
          + bi_ref[...])
    gs = (jnp.dot(sc * s_ref[...], ws, preferred_element_type=f32)
          + jnp.dot(g2[:, 64:], ws[:32, :], preferred_element_type=f32)
          + jnp.dot(g3, ws[32:, :], preferred_element_type=f32)
          + bs_ref[...])
    reset = jax.nn.sigmoid(gz[:, :128] + gs[:, :128])
    update = jax.nn.sigmoid(gz[:, 128:256] + gs[:, 128:256])
    cand = jnp.tanh(gz[:, 256:] + reset * gs[:, 256:])
    out_ref[...] = (1.0 - update) * cand + update * s_ref[...]


def _final(a0, a1, a2, a3, z, s, dinv, selfc, wi, ws, bi, bs):
    R = 2000
    grid = N // R
    return pl.pallas_call(
        _final_body,
        grid=(grid,),
        in_specs=[
            pl.BlockSpec((R, DF), lambda i: (i, 0)),
            pl.BlockSpec((R, DF), lambda i: (i, 0)),
            pl.BlockSpec((R, DF), lambda i: (i, 0)),
            pl.BlockSpec((R, DF), lambda i: (i, 0)),
            pl.BlockSpec((R, DZ), lambda i: (i, 0)),
            pl.BlockSpec((R, DS), lambda i: (i, 0)),
            pl.BlockSpec((R, 1), lambda i: (i, 0)),
            pl.BlockSpec((R, 1), lambda i: (i, 0)),
            pl.BlockSpec((DZ, 3 * DS), lambda i: (0, 0)),
            pl.BlockSpec((DS, 3 * DS), lambda i: (0, 0)),
            pl.BlockSpec((1, 3 * DS), lambda i: (0, 0)),
            pl.BlockSpec((1, 3 * DS), lambda i: (0, 0)),
        ],
        out_specs=pl.BlockSpec((R, DS), lambda i: (i, 0)),
        out_shape=jax.ShapeDtypeStruct((N, DS), jnp.float32),
    )(a0, a1, a2, a3, z, s, dinv, selfc, wi, ws, bi, bs)


def kernel(z, edge_index, s,
           Wi_reset, bi_reset, Ws_reset, bs_reset,
           Wi_update, bi_update, Ws_update, bs_update,
           Wi_cand, bi_cand, Ws_cand, bs_cand):
    src = edge_index[0]
    dst = edge_index[1]
    dst_deg = dst.reshape(32, NCH_DEG, CHUNK)
    src_agg = src.reshape(16, NCH_AGG, ACH)
    dst_agg = dst.reshape(16, NCH_AGG, ACH)

    deg_kernel, agg_kernel = _sc_kernels()
    deg_parts = deg_kernel(dst_deg)
    d0 = deg_parts[0, 0, :N].reshape(N, 1)
    d1 = deg_parts[1, 0, :N].reshape(N, 1)

    xs0, xs1, xs2, xs3, dinv, selfc = _prescale(d0, d1, z, s)
    a0, a1, a2, a3 = agg_kernel(xs0, xs1, xs2, xs3, src_agg, dst_agg)

    wi = jnp.concatenate([Wi_reset, Wi_update, Wi_cand], axis=1)
    ws = jnp.concatenate([Ws_reset, Ws_update, Ws_cand], axis=1)
    bi = jnp.concatenate([bi_reset, bi_update, bi_cand]).reshape(1, 3 * DS)
    bs = jnp.concatenate([bs_reset, bs_update, bs_cand]).reshape(1, 3 * DS)

    return _final(a0, a1, a2, a3, z, s, dinv, selfc, wi, ws, bi, bs)


# speedup vs baseline: 23.5125x; 1.0409x over previous
import functools

import jax
import jax.numpy as jnp
from jax import lax
from jax.experimental import pallas as pl
from jax.experimental.pallas import tpu as pltpu
from jax.experimental.pallas import tpu_sc as plsc

N = 10000
E = 320000
DZ = 256
DS = 128
DF = 96
CHUNK = 80
NCH_DEG = E // (32 * CHUNK)
ACH = 100
NBUF = 2
NCH_AGG = E // (16 * ACH)
NP = 10240
DEG_SL = 640
ROW_SL = 632
ROW_SL_LAST = N - 15 * ROW_SL


def _deg_body(dst_hbm, out_hbm, dst_v, ones_v, zeros_v, deg_sp):
    c = lax.axis_index("c")
    s = lax.axis_index("s")
    wid = c * 16 + s

    def fill(i, _):
        ones_v[pl.ds(i * 16, 16)] = jnp.full((16,), 1.0, jnp.float32)
        zeros_v[pl.ds(i * 16, 16)] = jnp.zeros((16,), jnp.float32)
        return 0
    lax.fori_loop(0, 5, fill, 0)

    def fillz(i, _):
        zeros_v[pl.ds(i * 16, 16)] = jnp.zeros((16,), jnp.float32)
        return 0
    lax.fori_loop(5, 40, fillz, 0)

    pltpu.sync_copy(zeros_v, deg_sp.at[pl.ds(s * DEG_SL, DEG_SL)])
    pltpu.sync_copy(dst_hbm.at[wid], dst_v)
    plsc.subcore_barrier()

    def body(j, _):
        pltpu.sync_copy(ones_v, deg_sp.at[dst_v.at[j]], add=True)
        return 0
    lax.fori_loop(0, NCH_DEG, body, 0)
    plsc.subcore_barrier()

    @pl.when(c == 0)
    def _():
        pltpu.sync_copy(deg_sp.at[pl.ds(s * DEG_SL, DEG_SL)],
                        out_hbm.at[0, 0, pl.ds(s * DEG_SL, DEG_SL)])

    @pl.when(c == 1)
    def _():
        pltpu.sync_copy(deg_sp.at[pl.ds(s * DEG_SL, DEG_SL)],
                        out_hbm.at[1, 0, pl.ds(s * DEG_SL, DEG_SL)])


def _agg_body(xs0, xs1, xs2, xs3, src_hbm, dst_hbm,
              out0, out1, out2, out3,
              src_v, dst_v, b0, b1, zrows_v,
              s0, s1, agg_sp):
    c = lax.axis_index("c")
    s = lax.axis_index("s")
    bufs = (b0, b1)
    sems = (s0, s1)

    def zrow(i, _):
        def zcol(k, _2):
            zrows_v[i, pl.ds(k * 16, 16)] = jnp.zeros((16,), jnp.float32)
            return 0
        lax.fori_loop(0, DF // 16, zcol, 0)
        return 0
    lax.fori_loop(0, ACH, zrow, 0)

    pltpu.sync_copy(src_hbm.at[s], src_v)
    pltpu.sync_copy(dst_hbm.at[s], dst_v)

    row0 = s * ROW_SL
    tables = (xs0, xs1, xs2, xs3)
    outs = (out0, out1, out2, out3)

    def zero_mine():
        @pl.when(s < 15)
        def _():
            for t in range(ROW_SL // ACH):
                pltpu.sync_copy(zrows_v,
                                agg_sp.at[pl.ds(row0 + t * ACH, ACH)])
            rem = ROW_SL % ACH
            pltpu.sync_copy(zrows_v.at[pl.ds(0, rem)],
                            agg_sp.at[pl.ds(row0 + ROW_SL - rem, rem)])

        @pl.when(s == 15)
        def _():
            base = 15 * ROW_SL
            for t in range(ROW_SL_LAST // ACH):
                pltpu.sync_copy(zrows_v,
                                agg_sp.at[pl.ds(base + t * ACH, ACH)])
            rem = ROW_SL_LAST % ACH
            pltpu.sync_copy(zrows_v.at[pl.ds(0, rem)],
                            agg_sp.at[pl.ds(base + ROW_SL_LAST - rem, rem)])

    def scatter_all(table):
        def body(g, _):
            base = g * NBUF
            hs = [pltpu.async_copy(table.at[src_v.at[base + i]],
                                   bufs[i], sems[i])
                  for i in range(NBUF)]
            for i in range(NBUF):
                hs[i].wait()
                pltpu.sync_copy(bufs[i], agg_sp.at[dst_v.at[base + i]],
                                add=True)
            return 0
        lax.fori_loop(0, NCH_AGG // NBUF, body, 0)

    def dump_mine(out):
        @pl.when(s < 15)
        def _():
            pltpu.sync_copy(agg_sp.at[pl.ds(row0, ROW_SL)],
                            out.at[pl.ds(row0, ROW_SL)])

        @pl.when(s == 15)
        def _():
            pltpu.sync_copy(agg_sp.at[pl.ds(15 * ROW_SL, ROW_SL_LAST)],
                            out.at[pl.ds(15 * ROW_SL, ROW_SL_LAST)])

    for p in range(2):
        zero_mine()
        plsc.subcore_barrier()

        @pl.when(c == 0)
        def _(p=p):
            scatter_all(tables[2 * p])

        @pl.when(c == 1)
        def _(p=p):
            scatter_all(tables[2 * p + 1])

        plsc.subcore_barrier()

        @pl.when(c == 0)
        def _(p=p):
            dump_mine(outs[2 * p])

        @pl.when(c == 1)
        def _(p=p):
            dump_mine(outs[2 * p + 1])

        if p == 0:
            plsc.subcore_barrier()


@functools.cache
def _sc_kernels():
    mesh = plsc.VectorSubcoreMesh(core_axis_name="c", subcore_axis_name="s")
    deg_k = functools.partial(
        pl.kernel,
        out_type=jax.ShapeDtypeStruct((2, 1, NP), jnp.float32),
        mesh=mesh,
        scratch_types=[
            pltpu.VMEM((NCH_DEG, CHUNK), jnp.int32),
            pltpu.VMEM((CHUNK,), jnp.float32),
            pltpu.VMEM((DEG_SL,), jnp.float32),
            pltpu.VMEM_SHARED((NP,), jnp.float32),
        ],
    )(_deg_body)
    agg_k = functools.partial(
        pl.kernel,
        out_type=[jax.ShapeDtypeStruct((N, DF), jnp.float32)] * 4,
        mesh=mesh,
        compiler_params=pltpu.CompilerParams(use_tc_tiling_on_sc=False),
        scratch_types=[
            pltpu.VMEM((NCH_AGG, ACH), jnp.int32),
            pltpu.VMEM((NCH_AGG, ACH), jnp.int32),
        ] + [pltpu.VMEM((ACH, DF), jnp.float32)] * (NBUF + 1)
        + [pltpu.SemaphoreType.DMA] * NBUF
        + [pltpu.VMEM_SHARED((N, DF), jnp.float32)],
    )(_agg_body)
    return deg_k, agg_k


def _prescale_body(d0, d1, z_ref, s_ref, xs0, xs1, xs2, xs3,
                   dinv_ref, selfc_ref):
    deg = d0[...] + d1[...] + 2.0
    di = lax.rsqrt(deg)
    dinv_ref[...] = di
    selfc_ref[...] = 2.0 / deg
    zs = z_ref[...] * di
    ss = s_ref[...] * di
    xs0[...] = zs[:, :96]
    xs1[...] = zs[:, 96:192]
    xs2[:, :64] = zs[:, 192:]
    xs2[:, 64:] = ss[:, :32]
    xs3[...] = ss[:, 32:]


def _prescale(d0, d1, z, s):
    R = 2000
    grid = N // R
    return pl.pallas_call(
        _prescale_body,
        grid=(grid,),
        in_specs=[
            pl.BlockSpec((R, 1), lambda i: (i, 0)),
            pl.BlockSpec((R, 1), lambda i: (i, 0)),
            pl.BlockSpec((R, DZ), lambda i: (i, 0)),
            pl.BlockSpec((R, DS), lambda i: (i, 0)),
        ],
        out_specs=[pl.BlockSpec((R, DF), lambda i: (i, 0))] * 4
        + [pl.BlockSpec((R, 1), lambda i: (i, 0))] * 2,
        out_shape=[jax.ShapeDtypeStruct((N, DF), jnp.float32)] * 4
        + [jax.ShapeDtypeStruct((N, 1), jnp.float32)] * 2,
    )(d0, d1, z, s)


def _final_body(a0, a1, a2, a3, z_ref, s_ref, dinv_ref, selfc_ref,
                wi_ref, ws_ref, bi_ref, bs_ref, out_ref):
    di = dinv_ref[...]
    sc = selfc_ref[...]
    g0 = di * a0[...]
    g1 = di * a1[...]
    g2 = di * a2[...]
    g3 = di * a3[...]
    wi = wi_ref[...]
    ws = ws_ref[...]
    f32 = jnp.float32
    gz = (jnp.dot(sc * z_ref[...], wi, preferred_element_type=f32)
          + jnp.dot(g0, wi[:96, :], preferred_element_type=f32)
          + jnp.dot(g1, wi[96:192, :], preferred_element_type=f32)
          + jnp.dot(g2[:, :64], wi[192:, :], preferred_element_type=f32)
          + bi_ref[...])
    gs = (jnp.dot(sc * s_ref[...], ws, preferred_element_type=f32)
          + jnp.dot(g2[:, 64:], ws[:32, :], preferred_element_type=f32)
          + jnp.dot(g3, ws[32:, :], preferred_element_type=f32)
          + bs_ref[...])
    reset = jax.nn.sigmoid(gz[:, :128] + gs[:, :128])
    update = jax.nn.sigmoid(gz[:, 128:256] + gs[:, 128:256])
    cand = jnp.tanh(gz[:, 256:] + reset * gs[:, 256:])
    out_ref[...] = (1.0 - update) * cand + update * s_ref[...]


def _final(a0, a1, a2, a3, z, s, dinv, selfc, wi, ws, bi, bs):
    R = 2000
    grid = N // R
    return pl.pallas_call(
        _final_body,
        grid=(grid,),
        in_specs=[
            pl.BlockSpec((R, DF), lambda i: (i, 0)),
            pl.BlockSpec((R, DF), lambda i: (i, 0)),
            pl.BlockSpec((R, DF), lambda i: (i, 0)),
            pl.BlockSpec((R, DF), lambda i: (i, 0)),
            pl.BlockSpec((R, DZ), lambda i: (i, 0)),
            pl.BlockSpec((R, DS), lambda i: (i, 0)),
            pl.BlockSpec((R, 1), lambda i: (i, 0)),
            pl.BlockSpec((R, 1), lambda i: (i, 0)),
            pl.BlockSpec((DZ, 3 * DS), lambda i: (0, 0)),
            pl.BlockSpec((DS, 3 * DS), lambda i: (0, 0)),
            pl.BlockSpec((1, 3 * DS), lambda i: (0, 0)),
            pl.BlockSpec((1, 3 * DS), lambda i: (0, 0)),
        ],
        out_specs=pl.BlockSpec((R, DS), lambda i: (i, 0)),
        out_shape=jax.ShapeDtypeStruct((N, DS), jnp.float32),
    )(a0, a1, a2, a3, z, s, dinv, selfc, wi, ws, bi, bs)


def kernel(z, edge_index, s,
           Wi_reset, bi_reset, Ws_reset, bs_reset,
           Wi_update, bi_update, Ws_update, bs_update,
           Wi_cand, bi_cand, Ws_cand, bs_cand):
    src = edge_index[0]
    dst = edge_index[1]
    dst_deg = dst.reshape(32, NCH_DEG, CHUNK)
    src_agg = src.reshape(16, NCH_AGG, ACH)
    dst_agg = dst.reshape(16, NCH_AGG, ACH)

    deg_kernel, agg_kernel = _sc_kernels()
    deg_parts = deg_kernel(dst_deg)
    d0 = deg_parts[0, 0, :N].reshape(N, 1)
    d1 = deg_parts[1, 0, :N].reshape(N, 1)

    xs0, xs1, xs2, xs3, dinv, selfc = _prescale(d0, d1, z, s)
    a0, a1, a2, a3 = agg_kernel(xs0, xs1, xs2, xs3, src_agg, dst_agg)

    wi = jnp.concatenate([Wi_reset, Wi_update, Wi_cand], axis=1)
    ws = jnp.concatenate([Ws_reset, Ws_update, Ws_cand], axis=1)
    bi = jnp.concatenate([bi_reset, bi_update, bi_cand]).reshape(1, 3 * DS)
    bs = jnp.concatenate([bs_reset, bs_update, bs_cand]).reshape(1, 3 * DS)

    return _final(a0, a1, a2, a3, z, s, dinv, selfc, wi, ws, bi, bs)


# speedup vs baseline: 24.3879x; 1.0372x over previous
import functools

import jax
import jax.numpy as jnp
from jax import lax
from jax.experimental import pallas as pl
from jax.experimental.pallas import tpu as pltpu
from jax.experimental.pallas import tpu_sc as plsc

N = 10000
E = 320000
DZ = 256
DS = 128
DF = 96
CHUNK = 80
NCH_DEG = E // (32 * CHUNK)
ACH = 125
NBUF = 2
NCH_AGG = E // (16 * ACH)
NP = 10240
DEG_SL = 640
ROW_SL = 632
ROW_SL_LAST = N - 15 * ROW_SL


def _deg_body(dst_hbm, out_hbm, dst_v, ones_v, zeros_v, deg_sp):
    c = lax.axis_index("c")
    s = lax.axis_index("s")
    wid = c * 16 + s

    def fill(i, _):
        ones_v[pl.ds(i * 16, 16)] = jnp.full((16,), 1.0, jnp.float32)
        zeros_v[pl.ds(i * 16, 16)] = jnp.zeros((16,), jnp.float32)
        return 0
    lax.fori_loop(0, 5, fill, 0)

    def fillz(i, _):
        zeros_v[pl.ds(i * 16, 16)] = jnp.zeros((16,), jnp.float32)
        return 0
    lax.fori_loop(5, 40, fillz, 0)

    pltpu.sync_copy(zeros_v, deg_sp.at[pl.ds(s * DEG_SL, DEG_SL)])
    pltpu.sync_copy(dst_hbm.at[wid], dst_v)
    plsc.subcore_barrier()

    def body(j, _):
        pltpu.sync_copy(ones_v, deg_sp.at[dst_v.at[j]], add=True)
        return 0
    lax.fori_loop(0, NCH_DEG, body, 0)
    plsc.subcore_barrier()

    @pl.when(c == 0)
    def _():
        pltpu.sync_copy(deg_sp.at[pl.ds(s * DEG_SL, DEG_SL)],
                        out_hbm.at[0, 0, pl.ds(s * DEG_SL, DEG_SL)])

    @pl.when(c == 1)
    def _():
        pltpu.sync_copy(deg_sp.at[pl.ds(s * DEG_SL, DEG_SL)],
                        out_hbm.at[1, 0, pl.ds(s * DEG_SL, DEG_SL)])


def _agg_body(xs0, xs1, xs2, xs3, src_hbm, dst_hbm,
              out0, out1, out2, out3,
              src_v, dst_v, b0, b1,
              s0, s1, agg_sp):
    c = lax.axis_index("c")
    s = lax.axis_index("s")
    bufs = (b0, b1)
    sems = (s0, s1)
    zrows_v = b0

    def fill_zeros():
        def zrow(i, _):
            def zcol(k, _2):
                zrows_v[i, pl.ds(k * 16, 16)] = jnp.zeros((16,), jnp.float32)
                return 0
            lax.fori_loop(0, DF // 16, zcol, 0)
            return 0
        lax.fori_loop(0, ACH, zrow, 0)

    pltpu.sync_copy(src_hbm.at[s], src_v)
    pltpu.sync_copy(dst_hbm.at[s], dst_v)

    row0 = s * ROW_SL
    tables = (xs0, xs1, xs2, xs3)
    outs = (out0, out1, out2, out3)

    def zero_mine():
        @pl.when(s < 15)
        def _():
            for t in range(ROW_SL // ACH):
                pltpu.sync_copy(zrows_v,
                                agg_sp.at[pl.ds(row0 + t * ACH, ACH)])
            rem = ROW_SL % ACH
            pltpu.sync_copy(zrows_v.at[pl.ds(0, rem)],
                            agg_sp.at[pl.ds(row0 + ROW_SL - rem, rem)])

        @pl.when(s == 15)
        def _():
            base = 15 * ROW_SL
            for t in range(ROW_SL_LAST // ACH):
                pltpu.sync_copy(zrows_v,
                                agg_sp.at[pl.ds(base + t * ACH, ACH)])
            rem = ROW_SL_LAST % ACH
            pltpu.sync_copy(zrows_v.at[pl.ds(0, rem)],
                            agg_sp.at[pl.ds(base + ROW_SL_LAST - rem, rem)])

    def scatter_all(table):
        def body(g, _):
            base = g * NBUF
            hs = [pltpu.async_copy(table.at[src_v.at[base + i]],
                                   bufs[i], sems[i])
                  for i in range(NBUF)]
            for i in range(NBUF):
                hs[i].wait()
                pltpu.sync_copy(bufs[i], agg_sp.at[dst_v.at[base + i]],
                                add=True)
            return 0
        lax.fori_loop(0, NCH_AGG // NBUF, body, 0)

    def dump_mine(out):
        @pl.when(s < 15)
        def _():
            pltpu.sync_copy(agg_sp.at[pl.ds(row0, ROW_SL)],
                            out.at[pl.ds(row0, ROW_SL)])

        @pl.when(s == 15)
        def _():
            pltpu.sync_copy(agg_sp.at[pl.ds(15 * ROW_SL, ROW_SL_LAST)],
                            out.at[pl.ds(15 * ROW_SL, ROW_SL_LAST)])

    for p in range(2):
        fill_zeros()
        zero_mine()
        plsc.subcore_barrier()

        @pl.when(c == 0)
        def _(p=p):
            scatter_all(tables[2 * p])

        @pl.when(c == 1)
        def _(p=p):
            scatter_all(tables[2 * p + 1])

        plsc.subcore_barrier()

        @pl.when(c == 0)
        def _(p=p):
            dump_mine(outs[2 * p])

        @pl.when(c == 1)
        def _(p=p):
            dump_mine(outs[2 * p + 1])

        if p == 0:
            plsc.subcore_barrier()


@functools.cache
def _sc_kernels():
    mesh = plsc.VectorSubcoreMesh(core_axis_name="c", subcore_axis_name="s")
    deg_k = functools.partial(
        pl.kernel,
        out_type=jax.ShapeDtypeStruct((2, 1, NP), jnp.float32),
        mesh=mesh,
        scratch_types=[
            pltpu.VMEM((NCH_DEG, CHUNK), jnp.int32),
            pltpu.VMEM((CHUNK,), jnp.float32),
            pltpu.VMEM((DEG_SL,), jnp.float32),
            pltpu.VMEM_SHARED((NP,), jnp.float32),
        ],
    )(_deg_body)
    agg_k = functools.partial(
        pl.kernel,
        out_type=[jax.ShapeDtypeStruct((N, DF), jnp.float32)] * 4,
        mesh=mesh,
        compiler_params=pltpu.CompilerParams(use_tc_tiling_on_sc=False),
        scratch_types=[
            pltpu.VMEM((NCH_AGG, ACH), jnp.int32),
            pltpu.VMEM((NCH_AGG, ACH), jnp.int32),
        ] + [pltpu.VMEM((ACH, DF), jnp.float32)] * NBUF
        + [pltpu.SemaphoreType.DMA] * NBUF
        + [pltpu.VMEM_SHARED((N, DF), jnp.float32)],
    )(_agg_body)
    return deg_k, agg_k


def _prescale_body(d0, d1, z_ref, s_ref, xs0, xs1, xs2, xs3,
                   dinv_ref, selfc_ref):
    deg = d0[...] + d1[...] + 2.0
    di = lax.rsqrt(deg)
    dinv_ref[...] = di
    selfc_ref[...] = 2.0 / deg
    zs = z_ref[...] * di
    ss = s_ref[...] * di
    xs0[...] = zs[:, :96]
    xs1[...] = zs[:, 96:192]
    xs2[:, :64] = zs[:, 192:]
    xs2[:, 64:] = ss[:, :32]
    xs3[...] = ss[:, 32:]


def _prescale(d0, d1, z, s):
    R = 2000
    grid = N // R
    return pl.pallas_call(
        _prescale_body,
        grid=(grid,),
        in_specs=[
            pl.BlockSpec((R, 1), lambda i: (i, 0)),
            pl.BlockSpec((R, 1), lambda i: (i, 0)),
            pl.BlockSpec((R, DZ), lambda i: (i, 0)),
            pl.BlockSpec((R, DS), lambda i: (i, 0)),
        ],
        out_specs=[pl.BlockSpec((R, DF), lambda i: (i, 0))] * 4
        + [pl.BlockSpec((R, 1), lambda i: (i, 0))] * 2,
        out_shape=[jax.ShapeDtypeStruct((N, DF), jnp.float32)] * 4
        + [jax.ShapeDtypeStruct((N, 1), jnp.float32)] * 2,
    )(d0, d1, z, s)


def _final_body(a0, a1, a2, a3, z_ref, s_ref, dinv_ref, selfc_ref,
                wi_ref, ws_ref, bi_ref, bs_ref, out_ref):
    di = dinv_ref[...]
    sc = selfc_ref[...]
    g0 = di * a0[...]
    g1 = di * a1[...]
    g2 = di * a2[...]
    g3 = di * a3[...]
    wi = wi_ref[...]
    ws = ws_ref[...]
    f32 = jnp.float32
    gz = (jnp.dot(sc * z_ref[...], wi, preferred_element_type=f32)
          + jnp.dot(g0, wi[:96, :], preferred_element_type=f32)
          + jnp.dot(g1, wi[96:192, :], preferred_element_type=f32)
          + jnp.dot(g2[:, :64], wi[192:, :], preferred_element_type=f32)
          + bi_ref[...])
    gs = (jnp.dot(sc * s_ref[...], ws, preferred_element_type=f32)
          + jnp.dot(g2[:, 64:], ws[:32, :], preferred_element_type=f32)
          + jnp.dot(g3, ws[32:, :], preferred_element_type=f32)
          + bs_ref[...])
    reset = jax.nn.sigmoid(gz[:, :128] + gs[:, :128])
    update = jax.nn.sigmoid(gz[:, 128:256] + gs[:, 128:256])
    cand = jnp.tanh(gz[:, 256:] + reset * gs[:, 256:])
    out_ref[...] = (1.0 - update) * cand + update * s_ref[...]


def _final(a0, a1, a2, a3, z, s, dinv, selfc, wi, ws, bi, bs):
    R = 2000
    grid = N // R
    return pl.pallas_call(
        _final_body,
        grid=(grid,),
        in_specs=[
            pl.BlockSpec((R, DF), lambda i: (i, 0)),
            pl.BlockSpec((R, DF), lambda i: (i, 0)),
            pl.BlockSpec((R, DF), lambda i: (i, 0)),
            pl.BlockSpec((R, DF), lambda i: (i, 0)),
            pl.BlockSpec((R, DZ), lambda i: (i, 0)),
            pl.BlockSpec((R, DS), lambda i: (i, 0)),
            pl.BlockSpec((R, 1), lambda i: (i, 0)),
            pl.BlockSpec((R, 1), lambda i: (i, 0)),
            pl.BlockSpec((DZ, 3 * DS), lambda i: (0, 0)),
            pl.BlockSpec((DS, 3 * DS), lambda i: (0, 0)),
            pl.BlockSpec((1, 3 * DS), lambda i: (0, 0)),
            pl.BlockSpec((1, 3 * DS), lambda i: (0, 0)),
        ],
        out_specs=pl.BlockSpec((R, DS), lambda i: (i, 0)),
        out_shape=jax.ShapeDtypeStruct((N, DS), jnp.float32),
    )(a0, a1, a2, a3, z, s, dinv, selfc, wi, ws, bi, bs)


def kernel(z, edge_index, s,
           Wi_reset, bi_reset, Ws_reset, bs_reset,
           Wi_update, bi_update, Ws_update, bs_update,
           Wi_cand, bi_cand, Ws_cand, bs_cand):
    src = edge_index[0]
    dst = edge_index[1]
    dst_deg = dst.reshape(32, NCH_DEG, CHUNK)
    src_agg = src.reshape(16, NCH_AGG, ACH)
    dst_agg = dst.reshape(16, NCH_AGG, ACH)

    deg_kernel, agg_kernel = _sc_kernels()
    deg_parts = deg_kernel(dst_deg)
    d0 = deg_parts[0, 0, :N].reshape(N, 1)
    d1 = deg_parts[1, 0, :N].reshape(N, 1)

    xs0, xs1, xs2, xs3, dinv, selfc = _prescale(d0, d1, z, s)
    a0, a1, a2, a3 = agg_kernel(xs0, xs1, xs2, xs3, src_agg, dst_agg)

    wi = jnp.concatenate([Wi_reset, Wi_update, Wi_cand], axis=1)
    ws = jnp.concatenate([Ws_reset, Ws_update, Ws_cand], axis=1)
    bi = jnp.concatenate([bi_reset, bi_update, bi_cand]).reshape(1, 3 * DS)
    bs = jnp.concatenate([bs_reset, bs_update, bs_cand]).reshape(1, 3 * DS)

    return _final(a0, a1, a2, a3, z, s, dinv, selfc, wi, ws, bi, bs)


# speedup vs baseline: 24.4894x; 1.0042x over previous
import functools

import jax
import jax.numpy as jnp
from jax import lax
from jax.experimental import pallas as pl
from jax.experimental.pallas import tpu as pltpu
from jax.experimental.pallas import tpu_sc as plsc

N = 10000
E = 320000
DZ = 256
DS = 128
DF = 96
ACH = 125
NBUF = 2
NCH = E // (16 * ACH)
NP = 10240
NSL = 640
RCH = 80
ROW_SL = 632
ROW_SL_LAST = N - 15 * ROW_SL


def _nr_rsqrt(v):
    i = lax.bitcast_convert_type(v, jnp.int32)
    i = jnp.int32(0x5F3759DF) - lax.shift_right_arithmetic(i, 1)
    y = lax.bitcast_convert_type(i, jnp.float32)
    for _ in range(3):
        y = y * (1.5 - 0.5 * v * y * y)
    return y


def _mega_body(z_hbm, s_hbm, src_hbm, dst_hbm,
               xs0, xs1, xs2, xs3, out0, out1, out2, out3,
               dinv_out, selfc_out,
               src_v, dst_v, b0, b1, ones_v, zeros_v,
               degb, dinvb, selfcb, sm0, sm1, deg_sp, agg_sp):
    c = lax.axis_index("c")
    s = lax.axis_index("s")
    bufs = (b0, b1)
    sems = (sm0, sm1)

    def fill(i, _):
        ones_v[pl.ds(i * 16, 16)] = jnp.full((16,), 1.0, jnp.float32)
        return 0
    lax.fori_loop(0, 8, fill, 0)

    def fillz(i, _):
        zeros_v[pl.ds(i * 16, 16)] = jnp.zeros((16,), jnp.float32)
        return 0
    lax.fori_loop(0, NSL // 16, fillz, 0)

    pltpu.sync_copy(src_hbm.at[s], src_v)
    pltpu.sync_copy(dst_hbm.at[s], dst_v)
    pltpu.sync_copy(zeros_v, deg_sp.at[pl.ds(s * NSL, NSL)])
    plsc.subcore_barrier()

    def dbody(j, _):
        pltpu.sync_copy(ones_v.at[pl.ds(0, ACH)],
                        deg_sp.at[dst_v.at[j]], add=True)
        return 0
    lax.fori_loop(0, NCH, dbody, 0)
    plsc.subcore_barrier()

    pltpu.sync_copy(deg_sp.at[pl.ds(s * NSL, NSL)], degb)

    def nbody(k, _):
        v = degb[pl.ds(k * 16, 16)] + 2.0
        dinvb[pl.ds(k * 16, 16)] = _nr_rsqrt(v)
        selfcb[pl.ds(k * 16, 16)] = 2.0 / v
        return 0
    lax.fori_loop(0, NSL // 16, nbody, 0)

    @pl.when(c == 0)
    def _():
        pltpu.sync_copy(dinvb, dinv_out.at[pl.ds(s * NSL, NSL)])
        pltpu.sync_copy(selfcb, selfc_out.at[pl.ds(s * NSL, NSL)])

    row0 = s * NSL
    nch_pre = jnp.where(s < 15, NSL // RCH, (N - 15 * NSL) // RCH)

    def prescale(loads, out):
        def pbody(k, _):
            r = row0 + k * RCH
            for (ref, col, bcol, w) in loads:
                pltpu.sync_copy(ref.at[pl.ds(r, RCH), pl.ds(col, w)],
                                b0.at[pl.ds(0, RCH), pl.ds(bcol, w)])

            def scale(m, _2):
                dv16 = dinvb[pl.ds(k * RCH + m * 16, 16)]
                for i in range(16):
                    r = m * 16 + i
                    dvs = dv16[i]
                    for f in range(DF // 16):
                        b0[r, pl.ds(f * 16, 16)] = (
                            b0[r, pl.ds(f * 16, 16)] * dvs)
                return 0
            lax.fori_loop(0, RCH // 16, scale, 0)
            pltpu.sync_copy(b0.at[pl.ds(0, RCH)], out.at[pl.ds(r, RCH)])
            return 0
        lax.fori_loop(0, nch_pre, pbody, 0)

    @pl.when(c == 0)
    def _():
        prescale(((z_hbm, 0, 0, DF),), xs0)
        prescale(((z_hbm, DF, 0, DF),), xs1)

    @pl.when(c == 1)
    def _():
        prescale(((z_hbm, 192, 0, 64), (s_hbm, 0, 64, 32)), xs2)
        prescale(((s_hbm, 32, 0, DF),), xs3)

    plsc.subcore_barrier()

    tables = (xs0, xs1, xs2, xs3)
    outs = (out0, out1, out2, out3)
    drow = s * ROW_SL

    def fill_zeros():
        def zrow(i, _):
            for f in range(DF // 16):
                b0[i, pl.ds(f * 16, 16)] = jnp.zeros((16,), jnp.float32)
            return 0
        lax.fori_loop(0, ACH, zrow, 0)

    def zero_mine():
        @pl.when(s < 15)
        def _():
            for t in range(ROW_SL // ACH):
                pltpu.sync_copy(b0, agg_sp.at[pl.ds(drow + t * ACH, ACH)])
            rem = ROW_SL % ACH
            pltpu.sync_copy(b0.at[pl.ds(0, rem)],
                            agg_sp.at[pl.ds(drow + ROW_SL - rem, rem)])

        @pl.when(s == 15)
        def _():
            base = 15 * ROW_SL
            for t in range(ROW_SL_LAST // ACH):
                pltpu.sync_copy(b0, agg_sp.at[pl.ds(base + t * ACH, ACH)])
            rem = ROW_SL_LAST % ACH
            pltpu.sync_copy(b0.at[pl.ds(0, rem)],
                            agg_sp.at[pl.ds(base + ROW_SL_LAST - rem, rem)])

    def scatter_all(table):
        def body(g, _):
            base = g * NBUF
            hs = [pltpu.async_copy(table.at[src_v.at[base + i]],
                                   bufs[i], sems[i])
                  for i in range(NBUF)]
            for i in range(NBUF):
                hs[i].wait()
                pltpu.sync_copy(bufs[i], agg_sp.at[dst_v.at[base + i]],
                                add=True)
            return 0
        lax.fori_loop(0, NCH // NBUF, body, 0)

    def dump_mine(out):
        @pl.when(s < 15)
        def _():
            pltpu.sync_copy(agg_sp.at[pl.ds(drow, ROW_SL)],
                            out.at[pl.ds(drow, ROW_SL)])

        @pl.when(s == 15)
        def _():
            pltpu.sync_copy(agg_sp.at[pl.ds(15 * ROW_SL, ROW_SL_LAST)],
                            out.at[pl.ds(15 * ROW_SL, ROW_SL_LAST)])

    for p in range(2):
        fill_zeros()
        zero_mine()
        plsc.subcore_barrier()

        @pl.when(c == 0)
        def _(p=p):
            scatter_all(tables[2 * p])

        @pl.when(c == 1)
        def _(p=p):
            scatter_all(tables[2 * p + 1])

        plsc.subcore_barrier()

        @pl.when(c == 0)
        def _(p=p):
            dump_mine(outs[2 * p])

        @pl.when(c == 1)
        def _(p=p):
            dump_mine(outs[2 * p + 1])

        if p == 0:
            plsc.subcore_barrier()


@functools.cache
def _sc_kernel():
    mesh = plsc.VectorSubcoreMesh(core_axis_name="c", subcore_axis_name="s")
    return functools.partial(
        pl.kernel,
        out_type=[jax.ShapeDtypeStruct((N, DF), jnp.float32)] * 8
        + [jax.ShapeDtypeStruct((NP,), jnp.float32)] * 2,
        mesh=mesh,
        compiler_params=pltpu.CompilerParams(use_tc_tiling_on_sc=False),
        scratch_types=[
            pltpu.VMEM((NCH, ACH), jnp.int32),
            pltpu.VMEM((NCH, ACH), jnp.int32),
            pltpu.VMEM((ACH, DF), jnp.float32),
            pltpu.VMEM((ACH, DF), jnp.float32),
            pltpu.VMEM((128,), jnp.float32),
            pltpu.VMEM((NSL,), jnp.float32),
            pltpu.VMEM((NSL,), jnp.float32),
            pltpu.VMEM((NSL,), jnp.float32),
            pltpu.VMEM((NSL,), jnp.float32),
            pltpu.SemaphoreType.DMA,
            pltpu.SemaphoreType.DMA,
            pltpu.VMEM_SHARED((NP,), jnp.float32),
            pltpu.VMEM_SHARED((N, DF), jnp.float32),
        ],
    )(_mega_body)


def _final_body(a0, a1, a2, a3, z_ref, s_ref, dinv_ref, selfc_ref,
                wi_ref, ws_ref, bi_ref, bs_ref, out_ref):
    di = dinv_ref[...]
    sc = selfc_ref[...]
    g0 = di * a0[...]
    g1 = di * a1[...]
    g2 = di * a2[...]
    g3 = di * a3[...]
    wi = wi_ref[...]
    ws = ws_ref[...]
    f32 = jnp.float32
    gz = (jnp.dot(sc * z_ref[...], wi, preferred_element_type=f32)
          + jnp.dot(g0, wi[:96, :], preferred_element_type=f32)
          + jnp.dot(g1, wi[96:192, :], preferred_element_type=f32)
          + jnp.dot(g2[:, :64], wi[192:, :], preferred_element_type=f32)
          + bi_ref[...])
    gs = (jnp.dot(sc * s_ref[...], ws, preferred_element_type=f32)
          + jnp.dot(g2[:, 64:], ws[:32, :], preferred_element_type=f32)
          + jnp.dot(g3, ws[32:, :], preferred_element_type=f32)
          + bs_ref[...])
    reset = jax.nn.sigmoid(gz[:, :128] + gs[:, :128])
    update = jax.nn.sigmoid(gz[:, 128:256] + gs[:, 128:256])
    cand = jnp.tanh(gz[:, 256:] + reset * gs[:, 256:])
    out_ref[...] = (1.0 - update) * cand + update * s_ref[...]


def _final(a0, a1, a2, a3, z, s, dinv, selfc, wi, ws, bi, bs):
    R = 2000
    grid = N // R
    return pl.pallas_call(
        _final_body,
        grid=(grid,),
        in_specs=[
            pl.BlockSpec((R, DF), lambda i: (i, 0)),
            pl.BlockSpec((R, DF), lambda i: (i, 0)),
            pl.BlockSpec((R, DF), lambda i: (i, 0)),
            pl.BlockSpec((R, DF), lambda i: (i, 0)),
            pl.BlockSpec((R, DZ), lambda i: (i, 0)),
            pl.BlockSpec((R, DS), lambda i: (i, 0)),
            pl.BlockSpec((R, 1), lambda i: (i, 0)),
            pl.BlockSpec((R, 1), lambda i: (i, 0)),
            pl.BlockSpec((DZ, 3 * DS), lambda i: (0, 0)),
            pl.BlockSpec((DS, 3 * DS), lambda i: (0, 0)),
            pl.BlockSpec((1, 3 * DS), lambda i: (0, 0)),
            pl.BlockSpec((1, 3 * DS), lambda i: (0, 0)),
        ],
        out_specs=pl.BlockSpec((R, DS), lambda i: (i, 0)),
        out_shape=jax.ShapeDtypeStruct((N, DS), jnp.float32),
    )(a0, a1, a2, a3, z, s, dinv, selfc, wi, ws, bi, bs)


def kernel(z, edge_index, s,
           Wi_reset, bi_reset, Ws_reset, bs_reset,
           Wi_update, bi_update, Ws_update, bs_update,
           Wi_cand, bi_cand, Ws_cand, bs_cand):
    src = edge_index[0]
    dst = edge_index[1]
    src_agg = src.reshape(16, NCH, ACH)
    dst_agg = dst.reshape(16, NCH, ACH)

    outs = _sc_kernel()(z, s, src_agg, dst_agg)
    a0, a1, a2, a3 = outs[4:8]
    dinv = outs[8][:N].reshape(N, 1)
    selfc = outs[9][:N].reshape(N, 1)

    wi = jnp.concatenate([Wi_reset, Wi_update, Wi_cand], axis=1)
    ws = jnp.concatenate([Ws_reset, Ws_update, Ws_cand], axis=1)
    bi = jnp.concatenate([bi_reset, bi_update, bi_cand]).reshape(1, 3 * DS)
    bs = jnp.concatenate([bs_reset, bs_update, bs_cand]).reshape(1, 3 * DS)

    return _final(a0, a1, a2, a3, z, s, dinv, selfc, wi, ws, bi, bs)


# speedup vs baseline: 24.9800x; 1.0200x over previous
import functools

import jax
import jax.numpy as jnp
from jax import lax
from jax.experimental import pallas as pl
from jax.experimental.pallas import tpu as pltpu
from jax.experimental.pallas import tpu_sc as plsc

N = 10000
E = 320000
DZ = 256
DS = 128
DF = 96
ACH = 125
NBUF = 2
NCH = E // (16 * ACH)
NP = 10240
NSL = 640
RCH = 80
ROW_SL = 632
ROW_SL_LAST = N - 15 * ROW_SL


def _nr_rsqrt(v):
    i = lax.bitcast_convert_type(v, jnp.int32)
    i = jnp.int32(0x5F3759DF) - lax.shift_right_arithmetic(i, 1)
    y = lax.bitcast_convert_type(i, jnp.float32)
    for _ in range(3):
        y = y * (1.5 - 0.5 * v * y * y)
    return y


def _mega_body(z_hbm, s_hbm, src_hbm, dst_hbm,
               xs0, xs1, xs2, xs3, out0, out1, out2, out3,
               dinv_out, selfc_out,
               src_v, dst_v, b0, b1, ones_v, zeros_v,
               degb, dinvb, selfcb, sm0, sm1, ss0, ss1, deg_sp, agg_sp):
    c = lax.axis_index("c")
    s = lax.axis_index("s")

    def fill(i, _):
        ones_v[pl.ds(i * 16, 16)] = jnp.full((16,), 1.0, jnp.float32)
        return 0
    lax.fori_loop(0, 8, fill, 0)

    def fillz(i, _):
        zeros_v[pl.ds(i * 16, 16)] = jnp.zeros((16,), jnp.float32)
        return 0
    lax.fori_loop(0, NSL // 16, fillz, 0)

    pltpu.sync_copy(src_hbm.at[s], src_v)
    pltpu.sync_copy(dst_hbm.at[s], dst_v)
    pltpu.sync_copy(zeros_v, deg_sp.at[pl.ds(s * NSL, NSL)])
    plsc.subcore_barrier()

    def dbody(j, _):
        pltpu.sync_copy(ones_v.at[pl.ds(0, ACH)],
                        deg_sp.at[dst_v.at[j]], add=True)
        return 0
    lax.fori_loop(0, NCH, dbody, 0)
    plsc.subcore_barrier()

    pltpu.sync_copy(deg_sp.at[pl.ds(s * NSL, NSL)], degb)

    def nbody(k, _):
        v = degb[pl.ds(k * 16, 16)] + 2.0
        dinvb[pl.ds(k * 16, 16)] = _nr_rsqrt(v)
        selfcb[pl.ds(k * 16, 16)] = 2.0 / v
        return 0
    lax.fori_loop(0, NSL // 16, nbody, 0)

    @pl.when(c == 0)
    def _():
        pltpu.sync_copy(dinvb, dinv_out.at[pl.ds(s * NSL, NSL)])
        pltpu.sync_copy(selfcb, selfc_out.at[pl.ds(s * NSL, NSL)])

    row0 = s * NSL
    nch_pre = jnp.where(s < 15, NSL // RCH, (N - 15 * NSL) // RCH)

    def prescale(loads, out):
        def pbody(k, _):
            r = row0 + k * RCH
            for (ref, col, bcol, w) in loads:
                pltpu.sync_copy(ref.at[pl.ds(r, RCH), pl.ds(col, w)],
                                b0.at[pl.ds(0, RCH), pl.ds(bcol, w)])

            def scale(m, _2):
                dv16 = dinvb[pl.ds(k * RCH + m * 16, 16)]
                for i in range(16):
                    r = m * 16 + i
                    dvs = dv16[i]
                    for f in range(DF // 16):
                        b0[r, pl.ds(f * 16, 16)] = (
                            b0[r, pl.ds(f * 16, 16)] * dvs)
                return 0
            lax.fori_loop(0, RCH // 16, scale, 0)
            pltpu.sync_copy(b0.at[pl.ds(0, RCH)], out.at[pl.ds(r, RCH)])
            return 0
        lax.fori_loop(0, nch_pre, pbody, 0)

    @pl.when(c == 0)
    def _():
        prescale(((z_hbm, 0, 0, DF),), xs0)
        prescale(((z_hbm, DF, 0, DF),), xs1)

    @pl.when(c == 1)
    def _():
        prescale(((z_hbm, 192, 0, 64), (s_hbm, 0, 64, 32)), xs2)
        prescale(((s_hbm, 32, 0, DF),), xs3)

    plsc.subcore_barrier()

    tables = (xs0, xs1, xs2, xs3)
    outs = (out0, out1, out2, out3)
    drow = s * ROW_SL

    def fill_zeros():
        def zrow(i, _):
            for f in range(DF // 16):
                b0[i, pl.ds(f * 16, 16)] = jnp.zeros((16,), jnp.float32)
            return 0
        lax.fori_loop(0, ACH, zrow, 0)

    def zero_mine():
        @pl.when(s < 15)
        def _():
            for t in range(ROW_SL // ACH):
                pltpu.sync_copy(b0, agg_sp.at[pl.ds(drow + t * ACH, ACH)])
            rem = ROW_SL % ACH
            pltpu.sync_copy(b0.at[pl.ds(0, rem)],
                            agg_sp.at[pl.ds(drow + ROW_SL - rem, rem)])

        @pl.when(s == 15)
        def _():
            base = 15 * ROW_SL
            for t in range(ROW_SL_LAST // ACH):
                pltpu.sync_copy(b0, agg_sp.at[pl.ds(base + t * ACH, ACH)])
            rem = ROW_SL_LAST % ACH
            pltpu.sync_copy(b0.at[pl.ds(0, rem)],
                            agg_sp.at[pl.ds(base + ROW_SL_LAST - rem, rem)])

    def scatter_all(table):
        def gstart(j, buf, sem):
            return pltpu.async_copy(table.at[src_v.at[j]], buf, sem)

        def sstart(j, buf, sem):
            return pltpu.async_copy(buf, agg_sp.at[dst_v.at[j]], sem,
                                    add=True)

        def body(g, _):
            j = 4 * g
            hg0 = gstart(j, b0, sm0)
            hg1 = gstart(j + 1, b1, sm1)
            hg0.wait()
            hs0 = sstart(j, b0, ss0)
            hg1.wait()
            hs1 = sstart(j + 1, b1, ss1)
            hs0.wait()
            hg2 = gstart(j + 2, b0, sm0)
            hs1.wait()
            hg3 = gstart(j + 3, b1, sm1)
            hg2.wait()
            hs2 = sstart(j + 2, b0, ss0)
            hg3.wait()
            hs3 = sstart(j + 3, b1, ss1)
            hs2.wait()
            hs3.wait()
            return 0
        lax.fori_loop(0, NCH // 4, body, 0)

    def dump_mine(out):
        @pl.when(s < 15)
        def _():
            pltpu.sync_copy(agg_sp.at[pl.ds(drow, ROW_SL)],
                            out.at[pl.ds(drow, ROW_SL)])

        @pl.when(s == 15)
        def _():
            pltpu.sync_copy(agg_sp.at[pl.ds(15 * ROW_SL, ROW_SL_LAST)],
                            out.at[pl.ds(15 * ROW_SL, ROW_SL_LAST)])

    for p in range(2):
        fill_zeros()
        zero_mine()
        plsc.subcore_barrier()

        @pl.when(c == 0)
        def _(p=p):
            scatter_all(tables[2 * p])

        @pl.when(c == 1)
        def _(p=p):
            scatter_all(tables[2 * p + 1])

        plsc.subcore_barrier()

        @pl.when(c == 0)
        def _(p=p):
            dump_mine(outs[2 * p])

        @pl.when(c == 1)
        def _(p=p):
            dump_mine(outs[2 * p + 1])

        if p == 0:
            plsc.subcore_barrier()


@functools.cache
def _sc_kernel():
    mesh = plsc.VectorSubcoreMesh(core_axis_name="c", subcore_axis_name="s")
    return functools.partial(
        pl.kernel,
        out_type=[jax.ShapeDtypeStruct((N, DF), jnp.float32)] * 8
        + [jax.ShapeDtypeStruct((NP,), jnp.float32)] * 2,
        mesh=mesh,
        compiler_params=pltpu.CompilerParams(use_tc_tiling_on_sc=False),
        scratch_types=[
            pltpu.VMEM((NCH, ACH), jnp.int32),
            pltpu.VMEM((NCH, ACH), jnp.int32),
            pltpu.VMEM((ACH, DF), jnp.float32),
            pltpu.VMEM((ACH, DF), jnp.float32),
            pltpu.VMEM((128,), jnp.float32),
            pltpu.VMEM((NSL,), jnp.float32),
            pltpu.VMEM((NSL,), jnp.float32),
            pltpu.VMEM((NSL,), jnp.float32),
            pltpu.VMEM((NSL,), jnp.float32),
            pltpu.SemaphoreType.DMA,
            pltpu.SemaphoreType.DMA,
            pltpu.SemaphoreType.DMA,
            pltpu.SemaphoreType.DMA,
            pltpu.VMEM_SHARED((NP,), jnp.float32),
            pltpu.VMEM_SHARED((N, DF), jnp.float32),
        ],
    )(_mega_body)


def _final_body(a0, a1, a2, a3, z_ref, s_ref, dinv_ref, selfc_ref,
                wi_ref, ws_ref, bi_ref, bs_ref, out_ref):
    di = dinv_ref[...]
    sc = selfc_ref[...]
    g0 = di * a0[...]
    g1 = di * a1[...]
    g2 = di * a2[...]
    g3 = di * a3[...]
    wi = wi_ref[...]
    ws = ws_ref[...]
    f32 = jnp.float32
    gz = (jnp.dot(sc * z_ref[...], wi, preferred_element_type=f32)
          + jnp.dot(g0, wi[:96, :], preferred_element_type=f32)
          + jnp.dot(g1, wi[96:192, :], preferred_element_type=f32)
          + jnp.dot(g2[:, :64], wi[192:, :], preferred_element_type=f32)
          + bi_ref[...])
    gs = (jnp.dot(sc * s_ref[...], ws, preferred_element_type=f32)
          + jnp.dot(g2[:, 64:], ws[:32, :], preferred_element_type=f32)
          + jnp.dot(g3, ws[32:, :], preferred_element_type=f32)
          + bs_ref[...])
    reset = jax.nn.sigmoid(gz[:, :128] + gs[:, :128])
    update = jax.nn.sigmoid(gz[:, 128:256] + gs[:, 128:256])
    cand = jnp.tanh(gz[:, 256:] + reset * gs[:, 256:])
    out_ref[...] = (1.0 - update) * cand + update * s_ref[...]


def _final(a0, a1, a2, a3, z, s, dinv, selfc, wi, ws, bi, bs):
    R = 2000
    grid = N // R
    return pl.pallas_call(
        _final_body,
        grid=(grid,),
        in_specs=[
            pl.BlockSpec((R, DF), lambda i: (i, 0)),
            pl.BlockSpec((R, DF), lambda i: (i, 0)),
            pl.BlockSpec((R, DF), lambda i: (i, 0)),
            pl.BlockSpec((R, DF), lambda i: (i, 0)),
            pl.BlockSpec((R, DZ), lambda i: (i, 0)),
            pl.BlockSpec((R, DS), lambda i: (i, 0)),
            pl.BlockSpec((R, 1), lambda i: (i, 0)),
            pl.BlockSpec((R, 1), lambda i: (i, 0)),
            pl.BlockSpec((DZ, 3 * DS), lambda i: (0, 0)),
            pl.BlockSpec((DS, 3 * DS), lambda i: (0, 0)),
            pl.BlockSpec((1, 3 * DS), lambda i: (0, 0)),
            pl.BlockSpec((1, 3 * DS), lambda i: (0, 0)),
        ],
        out_specs=pl.BlockSpec((R, DS), lambda i: (i, 0)),
        out_shape=jax.ShapeDtypeStruct((N, DS), jnp.float32),
    )(a0, a1, a2, a3, z, s, dinv, selfc, wi, ws, bi, bs)


def kernel(z, edge_index, s,
           Wi_reset, bi_reset, Ws_reset, bs_reset,
           Wi_update, bi_update, Ws_update, bs_update,
           Wi_cand, bi_cand, Ws_cand, bs_cand):
    src = edge_index[0]
    dst = edge_index[1]
    src_agg = src.reshape(16, NCH, ACH)
    dst_agg = dst.reshape(16, NCH, ACH)

    outs = _sc_kernel()(z, s, src_agg, dst_agg)
    a0, a1, a2, a3 = outs[4:8]
    dinv = outs[8][:N].reshape(N, 1)
    selfc = outs[9][:N].reshape(N, 1)

    wi = jnp.concatenate([Wi_reset, Wi_update, Wi_cand], axis=1)
    ws = jnp.concatenate([Ws_reset, Ws_update, Ws_cand], axis=1)
    bi = jnp.concatenate([bi_reset, bi_update, bi_cand]).reshape(1, 3 * DS)
    bs = jnp.concatenate([bs_reset, bs_update, bs_cand]).reshape(1, 3 * DS)

    return _final(a0, a1, a2, a3, z, s, dinv, selfc, wi, ws, bi, bs)


# speedup vs baseline: 25.2186x; 1.0096x over previous
import functools

import jax
import jax.numpy as jnp
from jax import lax
from jax.experimental import pallas as pl
from jax.experimental.pallas import tpu as pltpu
from jax.experimental.pallas import tpu_sc as plsc

N = 10000
E = 320000
DZ = 256
DS = 128
DF = 96
ACH = 125
NBUF = 2
NCH = E // (16 * ACH)
NP = 10240
NSL = 640
RCH = 80
ROW_SL = 632
ROW_SL_LAST = N - 15 * ROW_SL


def _nr_rsqrt(v):
    i = lax.bitcast_convert_type(v, jnp.int32)
    i = jnp.int32(0x5F3759DF) - lax.shift_right_arithmetic(i, 1)
    y = lax.bitcast_convert_type(i, jnp.float32)
    for _ in range(3):
        y = y * (1.5 - 0.5 * v * y * y)
    return y


def _mega_body(z_hbm, s_hbm, src_hbm, dst_hbm,
               xs0, xs1, xs2, xs3, out0, out1, out2, out3,
               dinv_out, selfc_out,
               src_v, dst_v, b0, b1, ones_v, zeros_v,
               degb, dinvb, selfcb, sm0, sm1, ss0, ss1, deg_sp, agg_sp):
    c = lax.axis_index("c")
    s = lax.axis_index("s")

    def fill(i, _):
        ones_v[pl.ds(i * 16, 16)] = jnp.full((16,), 1.0, jnp.float32)
        return 0
    lax.fori_loop(0, 8, fill, 0)

    def fillz(i, _):
        zeros_v[pl.ds(i * 16, 16)] = jnp.zeros((16,), jnp.float32)
        return 0
    lax.fori_loop(0, NSL // 16, fillz, 0)

    pltpu.sync_copy(src_hbm.at[s], src_v)
    pltpu.sync_copy(dst_hbm.at[s], dst_v)
    pltpu.sync_copy(zeros_v, deg_sp.at[pl.ds(s * NSL, NSL)])
    plsc.subcore_barrier()

    def dbody(g, _):
        h0 = pltpu.async_copy(ones_v.at[pl.ds(0, ACH)],
                              deg_sp.at[dst_v.at[2 * g]], ss0, add=True)
        h1 = pltpu.async_copy(ones_v.at[pl.ds(0, ACH)],
                              deg_sp.at[dst_v.at[2 * g + 1]], ss1, add=True)
        h0.wait()
        h1.wait()
        return 0
    lax.fori_loop(0, NCH // 2, dbody, 0)
    plsc.subcore_barrier()

    pltpu.sync_copy(deg_sp.at[pl.ds(s * NSL, NSL)], degb)

    def nbody(k, _):
        v = degb[pl.ds(k * 16, 16)] + 2.0
        dinvb[pl.ds(k * 16, 16)] = _nr_rsqrt(v)
        selfcb[pl.ds(k * 16, 16)] = 2.0 / v
        return 0
    lax.fori_loop(0, NSL // 16, nbody, 0)

    @pl.when(c == 0)
    def _():
        pltpu.sync_copy(dinvb, dinv_out.at[pl.ds(s * NSL, NSL)])
        pltpu.sync_copy(selfcb, selfc_out.at[pl.ds(s * NSL, NSL)])

    row0 = s * NSL
    nch_pre = jnp.where(s < 15, NSL // RCH, (N - 15 * NSL) // RCH)

    def prescale(loads, out):
        def pbody(k, _):
            r = row0 + k * RCH
            for (ref, col, bcol, w) in loads:
                pltpu.sync_copy(ref.at[pl.ds(r, RCH), pl.ds(col, w)],
                                b0.at[pl.ds(0, RCH), pl.ds(bcol, w)])

            def scale(m, _2):
                dv16 = dinvb[pl.ds(k * RCH + m * 16, 16)]
                for i in range(16):
                    r = m * 16 + i
                    dvs = dv16[i]
                    for f in range(DF // 16):
                        b0[r, pl.ds(f * 16, 16)] = (
                            b0[r, pl.ds(f * 16, 16)] * dvs)
                return 0
            lax.fori_loop(0, RCH // 16, scale, 0)
            pltpu.sync_copy(b0.at[pl.ds(0, RCH)], out.at[pl.ds(r, RCH)])
            return 0
        lax.fori_loop(0, nch_pre, pbody, 0)

    @pl.when(c == 0)
    def _():
        prescale(((z_hbm, 0, 0, DF),), xs0)
        prescale(((z_hbm, DF, 0, DF),), xs1)

    @pl.when(c == 1)
    def _():
        prescale(((z_hbm, 192, 0, 64), (s_hbm, 0, 64, 32)), xs2)
        prescale(((s_hbm, 32, 0, DF),), xs3)

    plsc.subcore_barrier()

    tables = (xs0, xs1, xs2, xs3)
    outs = (out0, out1, out2, out3)
    drow = s * ROW_SL

    def fill_zeros():
        def zrow(i, _):
            for f in range(DF // 16):
                b0[i, pl.ds(f * 16, 16)] = jnp.zeros((16,), jnp.float32)
            return 0
        lax.fori_loop(0, ACH, zrow, 0)

    def zero_mine():
        @pl.when(s < 15)
        def _():
            for t in range(ROW_SL // ACH):
                pltpu.sync_copy(b0, agg_sp.at[pl.ds(drow + t * ACH, ACH)])
            rem = ROW_SL % ACH
            pltpu.sync_copy(b0.at[pl.ds(0, rem)],
                            agg_sp.at[pl.ds(drow + ROW_SL - rem, rem)])

        @pl.when(s == 15)
        def _():
            base = 15 * ROW_SL
            for t in range(ROW_SL_LAST // ACH):
                pltpu.sync_copy(b0, agg_sp.at[pl.ds(base + t * ACH, ACH)])
            rem = ROW_SL_LAST % ACH
            pltpu.sync_copy(b0.at[pl.ds(0, rem)],
                            agg_sp.at[pl.ds(base + ROW_SL_LAST - rem, rem)])

    def scatter_all(table):
        def gstart(j, buf, sem):
            return pltpu.async_copy(table.at[src_v.at[j]], buf, sem)

        def sstart(j, buf, sem):
            return pltpu.async_copy(buf, agg_sp.at[dst_v.at[j]], sem,
                                    add=True)

        def body(g, _):
            j = 4 * g
            hg0 = gstart(j, b0, sm0)
            hg1 = gstart(j + 1, b1, sm1)
            hg0.wait()
            hs0 = sstart(j, b0, ss0)
            hg1.wait()
            hs1 = sstart(j + 1, b1, ss1)
            hs0.wait()
            hg2 = gstart(j + 2, b0, sm0)
            hs1.wait()
            hg3 = gstart(j + 3, b1, sm1)
            hg2.wait()
            hs2 = sstart(j + 2, b0, ss0)
            hg3.wait()
            hs3 = sstart(j + 3, b1, ss1)
            hs2.wait()
            hs3.wait()
            return 0
        lax.fori_loop(0, NCH // 4, body, 0)

    def dump_mine(out):
        @pl.when(s < 15)
        def _():
            pltpu.sync_copy(agg_sp.at[pl.ds(drow, ROW_SL)],
                            out.at[pl.ds(drow, ROW_SL)])

        @pl.when(s == 15)
        def _():
            pltpu.sync_copy(agg_sp.at[pl.ds(15 * ROW_SL, ROW_SL_LAST)],
                            out.at[pl.ds(15 * ROW_SL, ROW_SL_LAST)])

    for p in range(2):
        fill_zeros()
        zero_mine()
        plsc.subcore_barrier()

        @pl.when(c == 0)
        def _(p=p):
            scatter_all(tables[2 * p])

        @pl.when(c == 1)
        def _(p=p):
            scatter_all(tables[2 * p + 1])

        plsc.subcore_barrier()

        @pl.when(c == 0)
        def _(p=p):
            dump_mine(outs[2 * p])

        @pl.when(c == 1)
        def _(p=p):
            dump_mine(outs[2 * p + 1])

        if p == 0:
            plsc.subcore_barrier()


@functools.cache
def _sc_kernel():
    mesh = plsc.VectorSubcoreMesh(core_axis_name="c", subcore_axis_name="s")
    return functools.partial(
        pl.kernel,
        out_type=[jax.ShapeDtypeStruct((N, DF), jnp.float32)] * 8
        + [jax.ShapeDtypeStruct((NP,), jnp.float32)] * 2,
        mesh=mesh,
        compiler_params=pltpu.CompilerParams(use_tc_tiling_on_sc=False),
        scratch_types=[
            pltpu.VMEM((NCH, ACH), jnp.int32),
            pltpu.VMEM((NCH, ACH), jnp.int32),
            pltpu.VMEM((ACH, DF), jnp.float32),
            pltpu.VMEM((ACH, DF), jnp.float32),
            pltpu.VMEM((128,), jnp.float32),
            pltpu.VMEM((NSL,), jnp.float32),
            pltpu.VMEM((NSL,), jnp.float32),
            pltpu.VMEM((NSL,), jnp.float32),
            pltpu.VMEM((NSL,), jnp.float32),
            pltpu.SemaphoreType.DMA,
            pltpu.SemaphoreType.DMA,
            pltpu.SemaphoreType.DMA,
            pltpu.SemaphoreType.DMA,
            pltpu.VMEM_SHARED((NP,), jnp.float32),
            pltpu.VMEM_SHARED((N, DF), jnp.float32),
        ],
    )(_mega_body)


def _final_body(a0, a1, a2, a3, z_ref, s_ref, dinv_ref, selfc_ref,
                wi_ref, ws_ref, bi_ref, bs_ref, out_ref):
    di = dinv_ref[...]
    sc = selfc_ref[...]
    g0 = di * a0[...]
    g1 = di * a1[...]
    g2 = di * a2[...]
    g3 = di * a3[...]
    wi = wi_ref[...]
    ws = ws_ref[...]
    f32 = jnp.float32
    gz = (jnp.dot(sc * z_ref[...], wi, preferred_element_type=f32)
          + jnp.dot(g0, wi[:96, :], preferred_element_type=f32)
          + jnp.dot(g1, wi[96:192, :], preferred_element_type=f32)
          + jnp.dot(g2[:, :64], wi[192:, :], preferred_element_type=f32)
          + bi_ref[...])
    gs = (jnp.dot(sc * s_ref[...], ws, preferred_element_type=f32)
          + jnp.dot(g2[:, 64:], ws[:32, :], preferred_element_type=f32)
          + jnp.dot(g3, ws[32:, :], preferred_element_type=f32)
          + bs_ref[...])
    reset = jax.nn.sigmoid(gz[:, :128] + gs[:, :128])
    update = jax.nn.sigmoid(gz[:, 128:256] + gs[:, 128:256])
    cand = jnp.tanh(gz[:, 256:] + reset * gs[:, 256:])
    out_ref[...] = (1.0 - update) * cand + update * s_ref[...]


def _final(a0, a1, a2, a3, z, s, dinv, selfc, wi, ws, bi, bs):
    R = 2000
    grid = N // R
    return pl.pallas_call(
        _final_body,
        grid=(grid,),
        in_specs=[
            pl.BlockSpec((R, DF), lambda i: (i, 0)),
            pl.BlockSpec((R, DF), lambda i: (i, 0)),
            pl.BlockSpec((R, DF), lambda i: (i, 0)),
            pl.BlockSpec((R, DF), lambda i: (i, 0)),
            pl.BlockSpec((R, DZ), lambda i: (i, 0)),
            pl.BlockSpec((R, DS), lambda i: (i, 0)),
            pl.BlockSpec((R, 1), lambda i: (i, 0)),
            pl.BlockSpec((R, 1), lambda i: (i, 0)),
            pl.BlockSpec((DZ, 3 * DS), lambda i: (0, 0)),
            pl.BlockSpec((DS, 3 * DS), lambda i: (0, 0)),
            pl.BlockSpec((1, 3 * DS), lambda i: (0, 0)),
            pl.BlockSpec((1, 3 * DS), lambda i: (0, 0)),
        ],
        out_specs=pl.BlockSpec((R, DS), lambda i: (i, 0)),
        out_shape=jax.ShapeDtypeStruct((N, DS), jnp.float32),
    )(a0, a1, a2, a3, z, s, dinv, selfc, wi, ws, bi, bs)


def kernel(z, edge_index, s,
           Wi_reset, bi_reset, Ws_reset, bs_reset,
           Wi_update, bi_update, Ws_update, bs_update,
           Wi_cand, bi_cand, Ws_cand, bs_cand):
    src = edge_index[0]
    dst = edge_index[1]
    src_agg = src.reshape(16, NCH, ACH)
    dst_agg = dst.reshape(16, NCH, ACH)

    outs = _sc_kernel()(z, s, src_agg, dst_agg)
    a0, a1, a2, a3 = outs[4:8]
    dinv = outs[8][:N].reshape(N, 1)
    selfc = outs[9][:N].reshape(N, 1)

    wi = jnp.concatenate([Wi_reset, Wi_update, Wi_cand], axis=1)
    ws = jnp.concatenate([Ws_reset, Ws_update, Ws_cand], axis=1)
    bi = jnp.concatenate([bi_reset, bi_update, bi_cand]).reshape(1, 3 * DS)
    bs = jnp.concatenate([bs_reset, bs_update, bs_cand]).reshape(1, 3 * DS)

    return _final(a0, a1, a2, a3, z, s, dinv, selfc, wi, ws, bi, bs)


# speedup vs baseline: 25.2630x; 1.0018x over previous
import functools

import jax
import jax.numpy as jnp
from jax import lax
from jax.experimental import pallas as pl
from jax.experimental.pallas import tpu as pltpu
from jax.experimental.pallas import tpu_sc as plsc

N = 10000
E = 320000
DZ = 256
DS = 128
DF = 96
ACH = 125
NBUF = 2
NCH = E // (16 * ACH)
NP = 10240
NSL = 640
RCH = 80
ROW_SL = 632
ROW_SL_LAST = N - 15 * ROW_SL


def _nr_rsqrt(v):
    i = lax.bitcast_convert_type(v, jnp.int32)
    i = jnp.int32(0x5F3759DF) - lax.shift_right_arithmetic(i, 1)
    y = lax.bitcast_convert_type(i, jnp.float32)
    for _ in range(3):
        y = y * (1.5 - 0.5 * v * y * y)
    return y


def _mega_body(z_hbm, s_hbm, src_hbm, dst_hbm,
               xs0, xs1, xs2, xs3, out0, out1, out2, out3,
               dinv_out, selfc_out,
               src_v, dst_v, b0, b1, ones_v, zeros_v,
               degb, dinvb, selfcb, sm0, sm1, ss0, ss1, deg_sp, agg_sp):
    c = lax.axis_index("c")
    s = lax.axis_index("s")

    def fill(i, _):
        ones_v[pl.ds(i * 16, 16)] = jnp.full((16,), 1.0, jnp.float32)
        return 0
    lax.fori_loop(0, 8, fill, 0)

    def fillz(i, _):
        zeros_v[pl.ds(i * 16, 16)] = jnp.zeros((16,), jnp.float32)
        return 0
    lax.fori_loop(0, NSL // 16, fillz, 0)

    pltpu.sync_copy(src_hbm.at[s], src_v)
    pltpu.sync_copy(dst_hbm.at[s], dst_v)
    pltpu.sync_copy(zeros_v, deg_sp.at[pl.ds(s * NSL, NSL)])
    plsc.subcore_barrier()

    def dbody(g, _):
        h0 = pltpu.async_copy(ones_v.at[pl.ds(0, ACH)],
                              deg_sp.at[dst_v.at[2 * g]], ss0, add=True)
        h1 = pltpu.async_copy(ones_v.at[pl.ds(0, ACH)],
                              deg_sp.at[dst_v.at[2 * g + 1]], ss1, add=True)
        h0.wait()
        h1.wait()
        return 0
    lax.fori_loop(0, NCH // 2, dbody, 0)
    plsc.subcore_barrier()

    pltpu.sync_copy(deg_sp.at[pl.ds(s * NSL, NSL)], degb)

    def nbody(k, _):
        v = degb[pl.ds(k * 16, 16)] + 2.0
        dinvb[pl.ds(k * 16, 16)] = _nr_rsqrt(v)
        selfcb[pl.ds(k * 16, 16)] = 2.0 / v
        return 0
    lax.fori_loop(0, NSL // 16, nbody, 0)

    @pl.when(c == 0)
    def _():
        pltpu.sync_copy(dinvb, dinv_out.at[pl.ds(s * NSL, NSL)])
        pltpu.sync_copy(selfcb, selfc_out.at[pl.ds(s * NSL, NSL)])

    row0 = s * NSL
    nch_pre = jnp.where(s < 15, NSL // RCH, (N - 15 * NSL) // RCH)

    def prescale(loads, out):
        def pbody(k, _):
            r = row0 + k * RCH
            for (ref, col, bcol, w) in loads:
                pltpu.sync_copy(ref.at[pl.ds(r, RCH), pl.ds(col, w)],
                                b0.at[pl.ds(0, RCH), pl.ds(bcol, w)])

            def scale(m, _2):
                dv16 = dinvb[pl.ds(k * RCH + m * 16, 16)]
                for i in range(16):
                    r = m * 16 + i
                    dvs = dv16[i]
                    for f in range(DF // 16):
                        b0[r, pl.ds(f * 16, 16)] = (
                            b0[r, pl.ds(f * 16, 16)] * dvs)
                return 0
            lax.fori_loop(0, RCH // 16, scale, 0)
            pltpu.sync_copy(b0.at[pl.ds(0, RCH)], out.at[pl.ds(r, RCH)])
            return 0
        lax.fori_loop(0, nch_pre, pbody, 0)

    @pl.when(c == 0)
    def _():
        prescale(((z_hbm, 0, 0, DF),), xs0)
        prescale(((z_hbm, DF, 0, DF),), xs1)

    @pl.when(c == 1)
    def _():
        prescale(((z_hbm, 192, 0, 64), (s_hbm, 0, 64, 32)), xs2)
        prescale(((s_hbm, 32, 0, DF),), xs3)

    plsc.subcore_barrier()

    tables = (xs0, xs1, xs2, xs3)
    outs = (out0, out1, out2, out3)
    drow = s * ROW_SL

    def fill_zeros():
        def zrow(i, _):
            for f in range(DF // 16):
                b0[i, pl.ds(f * 16, 16)] = jnp.zeros((16,), jnp.float32)
            return 0
        lax.fori_loop(0, ACH, zrow, 0)

    def zero_mine():
        @pl.when(s < 15)
        def _():
            for t in range(ROW_SL // ACH):
                pltpu.sync_copy(b0, agg_sp.at[pl.ds(drow + t * ACH, ACH)])
            rem = ROW_SL % ACH
            pltpu.sync_copy(b0.at[pl.ds(0, rem)],
                            agg_sp.at[pl.ds(drow + ROW_SL - rem, rem)])

        @pl.when(s == 15)
        def _():
            base = 15 * ROW_SL
            for t in range(ROW_SL_LAST // ACH):
                pltpu.sync_copy(b0, agg_sp.at[pl.ds(base + t * ACH, ACH)])
            rem = ROW_SL_LAST % ACH
            pltpu.sync_copy(b0.at[pl.ds(0, rem)],
                            agg_sp.at[pl.ds(base + ROW_SL_LAST - rem, rem)])

    def scatter_all(table):
        def gstart(j, buf, sem):
            return pltpu.async_copy(table.at[src_v.at[j]], buf, sem)

        def sstart(j, buf, sem):
            return pltpu.async_copy(buf, agg_sp.at[dst_v.at[j]], sem,
                                    add=True)

        K = 4

        def body(g, _):
            j = 2 * K * g
            hg0 = gstart(j, b0, sm0)
            hg1 = gstart(j + 1, b1, sm1)
            for t in range(K):
                hg0.wait()
                hs0 = sstart(j + 2 * t, b0, ss0)
                hg1.wait()
                hs1 = sstart(j + 2 * t + 1, b1, ss1)
                hs0.wait()
                if t < K - 1:
                    hg0 = gstart(j + 2 * t + 2, b0, sm0)
                hs1.wait()
                if t < K - 1:
                    hg1 = gstart(j + 2 * t + 3, b1, sm1)
            return 0
        lax.fori_loop(0, NCH // (2 * K), body, 0)

    def dump_mine(out):
        @pl.when(s < 15)
        def _():
            pltpu.sync_copy(agg_sp.at[pl.ds(drow, ROW_SL)],
                            out.at[pl.ds(drow, ROW_SL)])

        @pl.when(s == 15)
        def _():
            pltpu.sync_copy(agg_sp.at[pl.ds(15 * ROW_SL, ROW_SL_LAST)],
                            out.at[pl.ds(15 * ROW_SL, ROW_SL_LAST)])

    for p in range(2):
        fill_zeros()
        zero_mine()
        plsc.subcore_barrier()

        @pl.when(c == 0)
        def _(p=p):
            scatter_all(tables[2 * p])

        @pl.when(c == 1)
        def _(p=p):
            scatter_all(tables[2 * p + 1])

        plsc.subcore_barrier()

        @pl.when(c == 0)
        def _(p=p):
            dump_mine(outs[2 * p])

        @pl.when(c == 1)
        def _(p=p):
            dump_mine(outs[2 * p + 1])

        if p == 0:
            plsc.subcore_barrier()


@functools.cache
def _sc_kernel():
    mesh = plsc.VectorSubcoreMesh(core_axis_name="c", subcore_axis_name="s")
    return functools.partial(
        pl.kernel,
        out_type=[jax.ShapeDtypeStruct((N, DF), jnp.float32)] * 8
        + [jax.ShapeDtypeStruct((NP,), jnp.float32)] * 2,
        mesh=mesh,
        compiler_params=pltpu.CompilerParams(use_tc_tiling_on_sc=False),
        scratch_types=[
            pltpu.VMEM((NCH, ACH), jnp.int32),
            pltpu.VMEM((NCH, ACH), jnp.int32),
            pltpu.VMEM((ACH, DF), jnp.float32),
            pltpu.VMEM((ACH, DF), jnp.float32),
            pltpu.VMEM((128,), jnp.float32),
            pltpu.VMEM((NSL,), jnp.float32),
            pltpu.VMEM((NSL,), jnp.float32),
            pltpu.VMEM((NSL,), jnp.float32),
            pltpu.VMEM((NSL,), jnp.float32),
            pltpu.SemaphoreType.DMA,
            pltpu.SemaphoreType.DMA,
            pltpu.SemaphoreType.DMA,
            pltpu.SemaphoreType.DMA,
            pltpu.VMEM_SHARED((NP,), jnp.float32),
            pltpu.VMEM_SHARED((N, DF), jnp.float32),
        ],
    )(_mega_body)


def _final_body(a0, a1, a2, a3, z_ref, s_ref, dinv_ref, selfc_ref,
                wi_ref, ws_ref, bi_ref, bs_ref, out_ref):
    di = dinv_ref[...]
    sc = selfc_ref[...]
    g0 = di * a0[...]
    g1 = di * a1[...]
    g2 = di * a2[...]
    g3 = di * a3[...]
    wi = wi_ref[...]
    ws = ws_ref[...]
    f32 = jnp.float32
    gz = (jnp.dot(sc * z_ref[...], wi, preferred_element_type=f32)
          + jnp.dot(g0, wi[:96, :], preferred_element_type=f32)
          + jnp.dot(g1, wi[96:192, :], preferred_element_type=f32)
          + jnp.dot(g2[:, :64], wi[192:, :], preferred_element_type=f32)
          + bi_ref[...])
    gs = (jnp.dot(sc * s_ref[...], ws, preferred_element_type=f32)
          + jnp.dot(g2[:, 64:], ws[:32, :], preferred_element_type=f32)
          + jnp.dot(g3, ws[32:, :], preferred_element_type=f32)
          + bs_ref[...])
    reset = jax.nn.sigmoid(gz[:, :128] + gs[:, :128])
    update = jax.nn.sigmoid(gz[:, 128:256] + gs[:, 128:256])
    cand = jnp.tanh(gz[:, 256:] + reset * gs[:, 256:])
    out_ref[...] = (1.0 - update) * cand + update * s_ref[...]


def _final(a0, a1, a2, a3, z, s, dinv, selfc, wi, ws, bi, bs):
    R = 2000
    grid = N // R
    return pl.pallas_call(
        _final_body,
        grid=(grid,),
        in_specs=[
            pl.BlockSpec((R, DF), lambda i: (i, 0)),
            pl.BlockSpec((R, DF), lambda i: (i, 0)),
            pl.BlockSpec((R, DF), lambda i: (i, 0)),
            pl.BlockSpec((R, DF), lambda i: (i, 0)),
            pl.BlockSpec((R, DZ), lambda i: (i, 0)),
            pl.BlockSpec((R, DS), lambda i: (i, 0)),
            pl.BlockSpec((R, 1), lambda i: (i, 0)),
            pl.BlockSpec((R, 1), lambda i: (i, 0)),
            pl.BlockSpec((DZ, 3 * DS), lambda i: (0, 0)),
            pl.BlockSpec((DS, 3 * DS), lambda i: (0, 0)),
            pl.BlockSpec((1, 3 * DS), lambda i: (0, 0)),
            pl.BlockSpec((1, 3 * DS), lambda i: (0, 0)),
        ],
        out_specs=pl.BlockSpec((R, DS), lambda i: (i, 0)),
        out_shape=jax.ShapeDtypeStruct((N, DS), jnp.float32),
    )(a0, a1, a2, a3, z, s, dinv, selfc, wi, ws, bi, bs)


def kernel(z, edge_index, s,
           Wi_reset, bi_reset, Ws_reset, bs_reset,
           Wi_update, bi_update, Ws_update, bs_update,
           Wi_cand, bi_cand, Ws_cand, bs_cand):
    src = edge_index[0]
    dst = edge_index[1]
    src_agg = src.reshape(16, NCH, ACH)
    dst_agg = dst.reshape(16, NCH, ACH)

    outs = _sc_kernel()(z, s, src_agg, dst_agg)
    a0, a1, a2, a3 = outs[4:8]
    dinv = outs[8][:N].reshape(N, 1)
    selfc = outs[9][:N].reshape(N, 1)

    wi = jnp.concatenate([Wi_reset, Wi_update, Wi_cand], axis=1)
    ws = jnp.concatenate([Ws_reset, Ws_update, Ws_cand], axis=1)
    bi = jnp.concatenate([bi_reset, bi_update, bi_cand]).reshape(1, 3 * DS)
    bs = jnp.concatenate([bs_reset, bs_update, bs_cand]).reshape(1, 3 * DS)

    return _final(a0, a1, a2, a3, z, s, dinv, selfc, wi, ws, bi, bs)
